# Initial kernel scaffold; baseline (speedup 1.0000x reference)
import jax
import jax.numpy as jnp
from jax.experimental import pallas as pl


def kernel(x, pos, edge_attr, temps, edge_index, triple_index, quadra_index, batch, W0, W12, Wres0, Wres12, A_src, A_dst, We, wd, wt, wq, M1, b1, M2, b2, M3, b3):
    raise NotImplementedError("write your pallas kernel here")


# speedup vs baseline: 17.5887x; 17.5887x over previous
import functools
from typing import Any

import jax
import jax.numpy as jnp
from jax import lax
from jax.experimental import pallas as pl
from jax.experimental.pallas import tpu as pltpu
from jax.experimental.pallas import tpu_sc as plsc

NC = 2
NS = 16
LN = 16
NW = NC * NS

_INTERPRET = False


def _mesh():
  return plsc.VectorSubcoreMesh(
      core_axis_name="c", subcore_axis_name="s", num_cores=NC,
      num_subcores=NS)


def _sc_params():
  return pltpu.CompilerParams(
      use_tc_tiling_on_sc=False, needs_layout_passes=False)


def _wid():
  return lax.axis_index("s") * NC + lax.axis_index("c")


def _iota16():
  return lax.iota(jnp.int32, 16)


def _splat(x):
  return jnp.full((16,), x, jnp.int32)


def _rsqrt(x):
  i = plsc.bitcast(x, jnp.int32)
  i = jnp.int32(0x5F3759DF) - lax.shift_right_arithmetic(i, 1)
  y = plsc.bitcast(i, jnp.float32)
  xh = x * jnp.float32(0.5)
  for _ in range(3):
    y = y * (jnp.float32(1.5) - xh * y * y)
  return y


def _sqrt16(x):
  return x * _rsqrt(x)


def _geom_body(n, e, t, q, c_sz,
               px, py, pz, src, dst, ti, tj, tk, qi, qj, qk, ql,
               dist_o, ts_part, qs_part,
               pxv, pyv, pzv, ts_acc, qs_acc, i0, i1, i2, i3, fb):
  w = _wid()
  it = _iota16()
  lane_masks = [it == j for j in range(16)]
  per = e // NW
  nch = per // c_sz
  ng = c_sz // 16

  pltpu.sync_copy(px, pxv)
  pltpu.sync_copy(py, pyv)
  pltpu.sync_copy(pz, pzv)

  def zero(ref, cnt):
    def zb(i, _):
      ref[pl.ds(i * 16, 16)] = jnp.zeros((16,), jnp.float32)
      return 0
    lax.fori_loop(0, cnt, zb, 0)

  zero(ts_acc, n // 16)
  zero(qs_acc, n // 16)

  def echunk(ci, _):
    base = w * per + ci * c_sz
    pltpu.sync_copy(src.at[pl.ds(base, c_sz)], i0)
    pltpu.sync_copy(dst.at[pl.ds(base, c_sz)], i1)

    def eg(g, _):
      s = i0[pl.ds(g * 16, 16)]
      d = i1[pl.ds(g * 16, 16)]
      dx = plsc.load_gather(pxv, [s]) - plsc.load_gather(pxv, [d])
      dy = plsc.load_gather(pyv, [s]) - plsc.load_gather(pyv, [d])
      dz = plsc.load_gather(pzv, [s]) - plsc.load_gather(pzv, [d])
      ss = dx * dx + dy * dy + dz * dz + jnp.float32(1e-8)
      fb[pl.ds(g * 16, 16)] = _sqrt16(ss)
      return 0
    lax.fori_loop(0, ng, eg, 0)
    pltpu.sync_copy(fb, dist_o.at[pl.ds(base, c_sz)])
    return 0
  lax.fori_loop(0, nch, echunk, 0)

  def tchunk(ci, _):
    base = w * per + ci * c_sz
    pltpu.sync_copy(ti.at[pl.ds(base, c_sz)], i0)
    pltpu.sync_copy(tj.at[pl.ds(base, c_sz)], i1)
    pltpu.sync_copy(tk.at[pl.ds(base, c_sz)], i2)

    def tg(g, _):
      a = i0[pl.ds(g * 16, 16)]
      b = i1[pl.ds(g * 16, 16)]
      c = i2[pl.ds(g * 16, 16)]
      bx = plsc.load_gather(pxv, [b])
      by = plsc.load_gather(pyv, [b])
      bz = plsc.load_gather(pzv, [b])
      v1x = plsc.load_gather(pxv, [a]) - bx
      v1y = plsc.load_gather(pyv, [a]) - by
      v1z = plsc.load_gather(pzv, [a]) - bz
      v2x = plsc.load_gather(pxv, [c]) - bx
      v2y = plsc.load_gather(pyv, [c]) - by
      v2z = plsc.load_gather(pzv, [c]) - bz
      dot = v1x * v2x + v1y * v2y + v1z * v2z
      s1 = v1x * v1x + v1y * v1y + v1z * v1z
      s2 = v2x * v2x + v2y * v2y + v2z * v2z
      cos = dot / (_sqrt16(s1 * s2) + jnp.float32(1e-8))
      for lm in lane_masks:
        plsc.addupdate_scatter(ts_acc, [b], cos, mask=lm)
      return 0
    lax.fori_loop(0, ng, tg, 0)
    return 0
  lax.fori_loop(0, nch, tchunk, 0)

  def qchunk(ci, _):
    base = w * per + ci * c_sz
    pltpu.sync_copy(qi.at[pl.ds(base, c_sz)], i0)
    pltpu.sync_copy(qj.at[pl.ds(base, c_sz)], i1)
    pltpu.sync_copy(qk.at[pl.ds(base, c_sz)], i2)
    pltpu.sync_copy(ql.at[pl.ds(base, c_sz)], i3)

    def qg(g, _):
      a = i0[pl.ds(g * 16, 16)]
      b = i1[pl.ds(g * 16, 16)]
      c = i2[pl.ds(g * 16, 16)]
      d = i3[pl.ds(g * 16, 16)]
      ax = plsc.load_gather(pxv, [a]); ay = plsc.load_gather(pyv, [a]); az = plsc.load_gather(pzv, [a])
      bx = plsc.load_gather(pxv, [b]); by = plsc.load_gather(pyv, [b]); bz = plsc.load_gather(pzv, [b])
      cx = plsc.load_gather(pxv, [c]); cy = plsc.load_gather(pyv, [c]); cz = plsc.load_gather(pzv, [c])
      ex = plsc.load_gather(pxv, [d]); ey = plsc.load_gather(pyv, [d]); ez = plsc.load_gather(pzv, [d])
      b1x = bx - ax; b1y = by - ay; b1z = bz - az
      b2x = cx - bx; b2y = cy - by; b2z = cz - bz
      b3x = ex - cx; b3y = ey - cy; b3z = ez - cz
      n1x = b1y * b2z - b1z * b2y
      n1y = b1z * b2x - b1x * b2z
      n1z = b1x * b2y - b1y * b2x
      n2x = b2y * b3z - b2z * b3y
      n2y = b2z * b3x - b2x * b3z
      n2z = b2x * b3y - b2y * b3x
      dot = n1x * n2x + n1y * n2y + n1z * n2z
      s1 = n1x * n1x + n1y * n1y + n1z * n1z
      s2 = n2x * n2x + n2y * n2y + n2z * n2z
      cos = dot / (_sqrt16(s1 * s2) + jnp.float32(1e-8))
      for lm in lane_masks:
        plsc.addupdate_scatter(qs_acc, [b], cos, mask=lm)
      return 0
    lax.fori_loop(0, ng, qg, 0)
    return 0
  lax.fori_loop(0, nch, qchunk, 0)

  pltpu.sync_copy(ts_acc, ts_part.at[w])
  pltpu.sync_copy(qs_acc, qs_part.at[w])


def _geom_call(pos3, src, dst, tix, tjx, tkx, qix, qjx, qkx, qlx):
  n = pos3.shape[0] // 3
  e = src.shape[0]
  c_sz = 80 if e % (NW * 80) == 0 else 16
  px = pos3[:n]
  py = pos3[n:2 * n]
  pz = pos3[2 * n:]
  body = functools.partial(_geom_body, n, e, tix.shape[0], qix.shape[0],
                           c_sz)
  f = pl.kernel(
      body,
      out_type=(
          jax.ShapeDtypeStruct((e,), jnp.float32),
          jax.ShapeDtypeStruct((NW, n), jnp.float32),
          jax.ShapeDtypeStruct((NW, n), jnp.float32),
      ),
      mesh=_mesh(),
      compiler_params=_sc_params(),
      scratch_types=(
          pltpu.VMEM((n,), jnp.float32),
          pltpu.VMEM((n,), jnp.float32),
          pltpu.VMEM((n,), jnp.float32),
          pltpu.VMEM((n,), jnp.float32),
          pltpu.VMEM((n,), jnp.float32),
          pltpu.VMEM((c_sz,), jnp.int32),
          pltpu.VMEM((c_sz,), jnp.int32),
          pltpu.VMEM((c_sz,), jnp.int32),
          pltpu.VMEM((c_sz,), jnp.int32),
          pltpu.VMEM((c_sz,), jnp.float32),
      ),
      interpret=_INTERPRET,
      name="sc_geom",
  )
  return f(px, py, pz, src, dst, tix, tjx, tkx, qix, qjx, qkx, qlx)


def _score_body(n, e, c_sz,
                stab, base_l, src, dst,
                ex_o, den_part,
                srows, drows, bbuf, exb, sidx, didx, den_acc, sem):
  w = _wid()
  it = _iota16()
  lo8 = it < 8
  hi8 = it >= 8
  pair = lax.shift_right_logical(it, 3)
  h8 = jnp.bitwise_and(it, 7)
  per = e // NW
  nch = per // c_sz

  def zero(i, _):
    den_acc[pl.ds(i * 16, 16)] = jnp.zeros((16,), jnp.float32)
    return 0
  lax.fori_loop(0, n * 8 // 16, zero, 0)

  def chunk(ci, _):
    base = w * per + ci * c_sz
    pltpu.sync_copy(src.at[pl.ds(base, c_sz)], sidx)
    pltpu.sync_copy(dst.at[pl.ds(base, c_sz)], didx)
    pltpu.async_copy(stab.at[sidx], srows, sem).wait()
    pltpu.async_copy(stab.at[didx], drows, sem).wait()
    pltpu.sync_copy(base_l.at[pl.ds(base, c_sz)], bbuf)

    def grp(g, _):
      rp = pair + 2 * g
      sv = plsc.load_gather(srows, [rp, h8])
      dv = plsc.load_gather(drows, [rp, h8 + 8])
      bv = plsc.load_gather(bbuf, [rp, h8])
      sc = sv + dv + bv
      sc = jnp.maximum(sc, sc * jnp.float32(0.2))
      ev = jnp.exp(sc)
      plsc.store_scatter(exb, [rp, h8], ev)
      dg = plsc.load_gather(didx, [pair + 2 * g])
      tgt = dg * 8 + h8
      plsc.addupdate_scatter(den_acc, [tgt], ev, mask=lo8)
      plsc.addupdate_scatter(den_acc, [tgt], ev, mask=hi8)
      return 0
    lax.fori_loop(0, c_sz // 2, grp, 0)
    pltpu.sync_copy(exb, ex_o.at[pl.ds(base, c_sz)])
    return 0
  lax.fori_loop(0, nch, chunk, 0)
  pltpu.sync_copy(den_acc, den_part.at[w])


def _score_call(stab, base_l, src, dst):
  n = stab.shape[0]
  e = src.shape[0]
  c_sz = 80 if e % (NW * 80) == 0 else 16
  body = functools.partial(_score_body, n, e, c_sz)
  f = pl.kernel(
      body,
      out_type=(
          jax.ShapeDtypeStruct((e, 8), jnp.float32),
          jax.ShapeDtypeStruct((NW, n * 8), jnp.float32),
      ),
      mesh=_mesh(),
      compiler_params=_sc_params(),
      scratch_types=(
          pltpu.VMEM((c_sz, 16), jnp.float32),
          pltpu.VMEM((c_sz, 16), jnp.float32),
          pltpu.VMEM((c_sz, 8), jnp.float32),
          pltpu.VMEM((c_sz, 8), jnp.float32),
          pltpu.VMEM((c_sz,), jnp.int32),
          pltpu.VMEM((c_sz,), jnp.int32),
          pltpu.VMEM((n * 8,), jnp.float32),
          pltpu.SemaphoreType.DMA,
      ),
      interpret=_INTERPRET,
      name="sc_score",
  )
  return f(stab, base_l, src, dst)


def _agg_body(n, e, c_sz,
              z2, ex, src, dst,
              num_o,
              zrows, exb, sidx, didx, zidx, zzero, acc, sem):
  c = lax.axis_index("c")
  s = lax.axis_index("s")
  it = _iota16()
  per = e // NS
  nch = per // c_sz
  rows_t = n // NS
  zch = 125 if rows_t % 125 == 0 else rows_t
  nzch = rows_t // zch

  zv = jnp.zeros((16,), jnp.float32)
  def zb(i, _):
    for j in range(8):
      plsc.store_scatter(zzero, [_splat(i), it + j * 16], zv)
    return 0
  lax.fori_loop(0, zch, zb, 0)
  def zdma(m, _):
    pltpu.sync_copy(zzero, acc.at[pl.ds(s * rows_t + m * zch, zch)])
    return 0
  lax.fori_loop(0, nzch, zdma, 0)
  plsc.subcore_barrier()

  def chunk(ci, _):
    base = s * per + ci * c_sz
    pltpu.sync_copy(src.at[pl.ds(base, c_sz)], sidx)
    pltpu.sync_copy(dst.at[pl.ds(base, c_sz)], didx)

    def adj(k, _):
      zidx[pl.ds(k * 16, 16)] = sidx[pl.ds(k * 16, 16)] + c * n
      return 0
    lax.fori_loop(0, c_sz // 16, adj, 0)
    pltpu.async_copy(z2.at[zidx], zrows, sem).wait()
    pltpu.sync_copy(ex.at[pl.ds(base, c_sz)], exb)

    def edge(k, _):
      for jh in range(4):
        hv = plsc.load_gather(exb, [_splat(k), _splat(0) + c * 4 + jh])
        for jj in range(2):
          j = jh * 2 + jj
          cp = it + j * 16
          row = plsc.load_gather(zrows, [_splat(k), cp])
          plsc.store_scatter(zrows, [_splat(k), cp], row * hv)
      return 0
    lax.fori_loop(0, c_sz, edge, 0)
    pltpu.sync_copy(zrows, acc.at[didx], add=True)
    return 0
  lax.fori_loop(0, nch, chunk, 0)
  plsc.subcore_barrier()

  def dump(m, _):
    off = s * rows_t + m * zch
    pltpu.sync_copy(acc.at[pl.ds(off, zch)],
                    num_o.at[pl.ds(c * n + off, zch)])
    return 0
  lax.fori_loop(0, nzch, dump, 0)


def _agg_call(z2, ex, src, dst):
  n = z2.shape[0] // 2
  e = src.shape[0]
  c_sz = 80 if e % (NS * 80) == 0 else 16
  rows_t = n // NS
  zch = 125 if rows_t % 125 == 0 else rows_t
  body = functools.partial(_agg_body, n, e, c_sz)
  f = pl.kernel(
      body,
      out_type=jax.ShapeDtypeStruct((2 * n, 128), jnp.float32),
      mesh=_mesh(),
      compiler_params=_sc_params(),
      scratch_types=(
          pltpu.VMEM((c_sz, 128), jnp.float32),
          pltpu.VMEM((c_sz, 8), jnp.float32),
          pltpu.VMEM((c_sz,), jnp.int32),
          pltpu.VMEM((c_sz,), jnp.int32),
          pltpu.VMEM((c_sz,), jnp.int32),
          pltpu.VMEM((zch, 128), jnp.float32),
          pltpu.VMEM_SHARED((n, 128), jnp.float32),
          pltpu.SemaphoreType.DMA,
      ),
      interpret=_INTERPRET,
      name="sc_agg",
  )
  return f(z2, ex, src, dst)


def _maxpool_body(n, g_cnt,
                  h2, batch,
                  maxpart,
                  acc, rows, bb, sem):
  c = lax.axis_index("c")
  s = lax.axis_index("s")
  w = _wid()
  it = _iota16()
  lane_masks = [it == j for j in range(16)]
  groups = n // 16
  gper = groups // NS
  rem = groups - gper * NS
  lo = s * gper + jnp.minimum(s, rem)
  cnt = gper + jnp.where(s < rem, 1, 0)

  ninf = jnp.full((16,), -1e30, jnp.float32)
  def ib(i, _):
    for j in range(8):
      plsc.store_scatter(acc, [_splat(i), it + j * 16], ninf)
    return 0
  lax.fori_loop(0, g_cnt, ib, 0)

  def grp(gg, _):
    pltpu.sync_copy(h2.at[pl.ds(c * n + gg * 16, 16)], rows)
    pltpu.sync_copy(batch.at[pl.ds(gg * 16, 16)], bb)
    bv = bb[:]
    for lane in range(16):
      gid = jnp.max(jnp.where(lane_masks[lane], bv, 0))
      for j in range(8):
        cp = it + j * 16
        cur = plsc.load_gather(acc, [_splat(gid), cp])
        row = plsc.load_gather(rows, [_splat(lane), cp])
        plsc.store_scatter(acc, [_splat(gid), cp], jnp.maximum(cur, row))
    return 0
  lax.fori_loop(lo, lo + cnt, grp, 0)
  pltpu.sync_copy(acc, maxpart.at[w])


def _maxpool_call(h2, batch, g_cnt):
  n = h2.shape[0] // 2
  body = functools.partial(_maxpool_body, n, g_cnt)
  f = pl.kernel(
      body,
      out_type=jax.ShapeDtypeStruct((NW, g_cnt, 128), jnp.float32),
      mesh=_mesh(),
      compiler_params=_sc_params(),
      scratch_types=(
          pltpu.VMEM((g_cnt, 128), jnp.float32),
          pltpu.VMEM((16, 128), jnp.float32),
          pltpu.VMEM((16,), jnp.int32),
          pltpu.SemaphoreType.DMA,
      ),
      interpret=_INTERPRET,
      name="sc_maxpool",
  )
  return f(h2, batch)


def _tc_call(body, grid, in_specs, out_specs, out_shape, name):
  return pl.pallas_call(
      body,
      grid=grid,
      in_specs=in_specs,
      out_specs=out_specs,
      out_shape=out_shape,
      interpret=_INTERPRET,
      name=name,
  )


def _prep_base(edge_attr, dist, we_all, wd_all):
  e = edge_attr.shape[0]
  be = min(e, 2048)
  nl = we_all.shape[1] // 8

  def body(ea_ref, d_ref, we_ref, wd_ref, o_ref):
    v = jnp.dot(ea_ref[...], we_ref[...],
                preferred_element_type=jnp.float32)
    v = v + d_ref[...] * wd_ref[...]
    for l in range(nl):
      o_ref[l] = v[:, 8 * l:8 * l + 8]

  return _tc_call(
      body, (pl.cdiv(e, be),),
      [
          pl.BlockSpec((be, 16), lambda i: (i, 0)),
          pl.BlockSpec((be, 1), lambda i: (i, 0)),
          pl.BlockSpec((16, 8 * nl), lambda i: (0, 0)),
          pl.BlockSpec((1, 8 * nl), lambda i: (0, 0)),
      ],
      pl.BlockSpec((nl, be, 8), lambda i: (0, i, 0)),
      jax.ShapeDtypeStruct((nl, e, 8), jnp.float32),
      "tc_prep_base")(edge_attr, dist, we_all, wd_all)


def _layer_head(h, w_mat, wr_mat, sa_mat, extras=None):
  n, din = h.shape
  bn = min(n, 2048)
  hid = w_mat.shape[1]
  with_parts = extras is not None

  def body(*refs):
    if with_parts:
      h_ref, w_ref, wr_ref, sa_ref, tp_ref, qp_ref, ones_ref, \
          z2_ref, s_ref, r_ref, ts_ref, qs_ref = refs
    else:
      h_ref, w_ref, wr_ref, sa_ref, z2_ref, s_ref, r_ref = refs
    z = jnp.dot(h_ref[...], w_ref[...], preferred_element_type=jnp.float32)
    z2_ref[0] = z[:, :hid // 2]
    z2_ref[1] = z[:, hid // 2:]
    s_ref[...] = jnp.dot(z, sa_ref[...], preferred_element_type=jnp.float32)
    r_ref[...] = jnp.dot(h_ref[...], wr_ref[...],
                         preferred_element_type=jnp.float32)
    if with_parts:
      ts_ref[...] = lax.dot_general(
          tp_ref[...], ones_ref[...], (((0,), (0,)), ((), ())),
          preferred_element_type=jnp.float32)
      qs_ref[...] = lax.dot_general(
          qp_ref[...], ones_ref[...], (((0,), (0,)), ((), ())),
          preferred_element_type=jnp.float32)

  in_specs = [
      pl.BlockSpec((bn, din), lambda i: (i, 0)),
      pl.BlockSpec((din, hid), lambda i: (0, 0)),
      pl.BlockSpec((din, hid), lambda i: (0, 0)),
      pl.BlockSpec((hid, 16), lambda i: (0, 0)),
  ]
  args = [h, w_mat, wr_mat, sa_mat]
  out_specs = [
      pl.BlockSpec((2, bn, hid // 2), lambda i: (0, i, 0)),
      pl.BlockSpec((bn, 16), lambda i: (i, 0)),
      pl.BlockSpec((bn, hid), lambda i: (i, 0)),
  ]
  out_shape = [
      jax.ShapeDtypeStruct((2, n, hid // 2), jnp.float32),
      jax.ShapeDtypeStruct((n, 16), jnp.float32),
      jax.ShapeDtypeStruct((n, hid), jnp.float32),
  ]
  if with_parts:
    ts_part, qs_part = extras
    nwp = ts_part.shape[0]
    in_specs += [
        pl.BlockSpec((nwp, bn), lambda i: (0, i)),
        pl.BlockSpec((nwp, bn), lambda i: (0, i)),
        pl.BlockSpec((nwp, 1), lambda i: (0, 0)),
    ]
    args += [ts_part, qs_part, jnp.ones((nwp, 1), jnp.float32)]
    out_specs += [
        pl.BlockSpec((bn, 1), lambda i: (i, 0)),
        pl.BlockSpec((bn, 1), lambda i: (i, 0)),
    ]
    out_shape += [
        jax.ShapeDtypeStruct((n, 1), jnp.float32),
        jax.ShapeDtypeStruct((n, 1), jnp.float32),
    ]
  return _tc_call(body, (pl.cdiv(n, bn),), in_specs, out_specs, out_shape,
                  "tc_layer_head")(*args)


def _den_reduce(den_part_flat):
  nwp, m = den_part_flat.shape
  bm = 8192 if m > 8192 else m

  def body(dp_ref, o_ref):
    o_ref[...] = jnp.sum(dp_ref[...], axis=0, keepdims=True)

  return _tc_call(
      body, (pl.cdiv(m, bm),),
      [pl.BlockSpec((nwp, bm), lambda i: (0, i))],
      pl.BlockSpec((1, bm), lambda i: (0, i)),
      jax.ShapeDtypeStruct((1, m), jnp.float32),
      "tc_den_reduce")(den_part_flat)


def _node_update(num2, den, r, ts, qs, wt_l, wq_l, ones_exp):
  n = r.shape[0]
  hid = r.shape[1]
  bn = min(n, 2048)

  def body(num_ref, den_ref, r_ref, ts_ref, qs_ref, wt_ref, wq_ref,
           oe_ref, h_ref):
    num = jnp.concatenate([num_ref[0], num_ref[1]], axis=-1)
    den = den_ref[...] + jnp.float32(1e-9)
    den_e = jnp.dot(den, oe_ref[...], preferred_element_type=jnp.float32)
    v = num / den_e + r_ref[...] + ts_ref[...] * wt_ref[...] \
        + qs_ref[...] * wq_ref[...]
    h_ref[...] = jnp.where(v > 0, v, jnp.exp(jnp.minimum(v, 0.0)) - 1.0)

  return _tc_call(
      body, (pl.cdiv(n, bn),),
      [
          pl.BlockSpec((2, bn, hid // 2), lambda i: (0, i, 0)),
          pl.BlockSpec((bn, 8), lambda i: (i, 0)),
          pl.BlockSpec((bn, hid), lambda i: (i, 0)),
          pl.BlockSpec((bn, 1), lambda i: (i, 0)),
          pl.BlockSpec((bn, 1), lambda i: (i, 0)),
          pl.BlockSpec((1, hid), lambda i: (0, 0)),
          pl.BlockSpec((1, hid), lambda i: (0, 0)),
          pl.BlockSpec((8, hid), lambda i: (0, 0)),
      ],
      pl.BlockSpec((bn, hid), lambda i: (i, 0)),
      jax.ShapeDtypeStruct((n, hid), jnp.float32),
      "tc_node_update")(num2, den, r, ts, qs, wt_l, wq_l, ones_exp)


def _split_rows(h):
  n, hid = h.shape
  bn = min(n, 2048)

  def body(h_ref, o_ref):
    o_ref[0] = h_ref[:, :hid // 2]
    o_ref[1] = h_ref[:, hid // 2:]

  return _tc_call(
      body, (pl.cdiv(n, bn),),
      [pl.BlockSpec((bn, hid), lambda i: (i, 0))],
      pl.BlockSpec((2, bn, hid // 2), lambda i: (0, i, 0)),
      jax.ShapeDtypeStruct((2, n, hid // 2), jnp.float32),
      "tc_split")(h)


def _sum_pool(h, batch_col, g_cnt):
  n, hid = h.shape
  bn = min(n, 2048)

  def body(h_ref, b_ref, o_ref):
    @pl.when(pl.program_id(0) == 0)
    def _():
      o_ref[...] = jnp.zeros_like(o_ref)
    valid = n - pl.program_id(0) * bn
    rows = lax.broadcasted_iota(jnp.int32, (bn, g_cnt), 0)
    onehot = (b_ref[...] == lax.broadcasted_iota(jnp.int32, (bn, g_cnt), 1)
              ).astype(jnp.float32)
    onehot = jnp.where(rows < valid, onehot, 0.0)
    o_ref[...] += lax.dot_general(
        onehot, h_ref[...], (((0,), (0,)), ((), ())),
        preferred_element_type=jnp.float32)

  return _tc_call(
      body, (pl.cdiv(n, bn),),
      [
          pl.BlockSpec((bn, hid), lambda i: (i, 0)),
          pl.BlockSpec((bn, 1), lambda i: (i, 0)),
      ],
      pl.BlockSpec((g_cnt, hid), lambda i: (0, 0)),
      jax.ShapeDtypeStruct((g_cnt, hid), jnp.float32),
      "tc_sum_pool")(h, batch_col)


def _readout(sum_pool, maxpart, temps, m1a, m1b, m1c, b1, m2, b2, m3, b3):
  g_cnt = sum_pool.shape[0]
  nwp = maxpart.shape[0]

  def body(sp_ref, mp_ref, t_ref, m1a_ref, m1b_ref, m1c_ref, b1_ref,
           m2_ref, b2_ref, m3_ref, b3_ref, o_ref):
    mp0 = mp_ref[0]
    mp1 = mp_ref[1]
    for k in range(2, nwp, 2):
      mp0 = jnp.maximum(mp0, mp_ref[k])
      mp1 = jnp.maximum(mp1, mp_ref[k + 1])
    mp = jnp.concatenate([mp0, mp1], axis=-1)
    mp = jnp.where(mp <= -1e29, 0.0, mp)
    u = jnp.dot(sp_ref[...], m1a_ref[...], preferred_element_type=jnp.float32)
    u += jnp.dot(mp, m1b_ref[...], preferred_element_type=jnp.float32)
    u += jnp.dot(t_ref[...], m1c_ref[...], preferred_element_type=jnp.float32)
    u += b1_ref[...]
    u = jnp.where(u > 0, u, jnp.exp(jnp.minimum(u, 0.0)) - 1.0)
    u = jnp.dot(u, m2_ref[...], preferred_element_type=jnp.float32) + b2_ref[...]
    u = jnp.where(u > 0, u, jnp.exp(jnp.minimum(u, 0.0)) - 1.0)
    o_ref[...] = jnp.dot(u, m3_ref[...],
                         preferred_element_type=jnp.float32) + b3_ref[...]

  hid = sum_pool.shape[1]
  return _tc_call(
      body, (1,),
      [
          pl.BlockSpec((g_cnt, hid), lambda i: (0, 0)),
          pl.BlockSpec((nwp, g_cnt, hid // 2), lambda i: (0, 0, 0)),
          pl.BlockSpec((g_cnt, 1), lambda i: (0, 0)),
          pl.BlockSpec((hid, 256), lambda i: (0, 0)),
          pl.BlockSpec((hid, 256), lambda i: (0, 0)),
          pl.BlockSpec((1, 256), lambda i: (0, 0)),
          pl.BlockSpec((1, 256), lambda i: (0, 0)),
          pl.BlockSpec((256, 256), lambda i: (0, 0)),
          pl.BlockSpec((1, 256), lambda i: (0, 0)),
          pl.BlockSpec((256, 1), lambda i: (0, 0)),
          pl.BlockSpec((1, 1), lambda i: (0, 0)),
      ],
      pl.BlockSpec((g_cnt, 1), lambda i: (0, 0)),
      jax.ShapeDtypeStruct((g_cnt, 1), jnp.float32),
      "tc_readout")(sum_pool, maxpart, temps, m1a, m1b, m1c, b1, m2, b2,
                    m3, b3)


def _block_diag_heads(a):
  h, dh = a.shape
  eye = jnp.eye(h, dtype=a.dtype)
  return (eye[:, None, :] * a[:, :, None]).reshape(h * dh, h)


def kernel(x, pos, edge_attr, temps, edge_index, triple_index, quadra_index,
           batch, W0, W12, Wres0, Wres12, A_src, A_dst, We, wd, wt, wq,
           M1, b1, M2, b2, M3, b3):
  n = x.shape[0]
  g_cnt = temps.shape[0]
  hid = W0.shape[1]
  i32 = jnp.int32

  src = edge_index[0].astype(i32)
  dst = edge_index[1].astype(i32)
  tix = triple_index[0].astype(i32)
  tjx = triple_index[1].astype(i32)
  tkx = triple_index[2].astype(i32)
  qix = quadra_index[0].astype(i32)
  qjx = quadra_index[1].astype(i32)
  qkx = quadra_index[2].astype(i32)
  qlx = quadra_index[3].astype(i32)
  batch_i = batch.astype(i32)
  pos3 = jnp.concatenate([pos[:, 0], pos[:, 1], pos[:, 2]])

  dist, ts_part, qs_part = _geom_call(pos3, src, dst, tix, tjx, tkx,
                                      qix, qjx, qkx, qlx)

  we_all = jnp.transpose(We, (1, 0, 2)).reshape(16, -1)
  wd_all = wd.reshape(1, -1)
  base = _prep_base(edge_attr, dist.reshape(-1, 1), we_all, wd_all)

  ones_exp = jnp.repeat(jnp.eye(8, dtype=jnp.float32), hid // 8, axis=1)

  ws = [W0, W12[0], W12[1]]
  wrs = [Wres0, Wres12[0], Wres12[1]]
  sa_mats = [jnp.concatenate(
      [_block_diag_heads(A_src[l]), _block_diag_heads(A_dst[l])], axis=1)
      for l in range(3)]

  h = x
  ts = qs = None
  for l in range(3):
    if l == 0:
      z2, stab, r, ts, qs = _layer_head(h, ws[l], wrs[l], sa_mats[l],
                                        extras=(ts_part, qs_part))
    else:
      z2, stab, r = _layer_head(h, ws[l], wrs[l], sa_mats[l])
    ex, den_part = _score_call(stab, base[l], src, dst)
    num2 = _agg_call(z2.reshape(2 * n, hid // 2), ex, src, dst)
    den = _den_reduce(den_part).reshape(n, 8)
    h = _node_update(num2.reshape(2, n, hid // 2), den, r, ts, qs,
                     wt[l].reshape(1, hid), wq[l].reshape(1, hid), ones_exp)

  h2 = _split_rows(h)
  maxpart = _maxpool_call(h2.reshape(2 * n, hid // 2), batch_i, g_cnt)
  sp = _sum_pool(h, batch_i.reshape(n, 1), g_cnt)
  out = _readout(sp, maxpart, temps, M1[:hid], M1[hid:2 * hid],
                 M1[2 * hid:], b1.reshape(1, -1), M2, b2.reshape(1, -1),
                 M3, b3.reshape(1, 1))
  return out


# speedup vs baseline: 18.4176x; 1.0471x over previous
import functools
from typing import Any

import jax
import jax.numpy as jnp
from jax import lax
from jax.experimental import pallas as pl
from jax.experimental.pallas import tpu as pltpu
from jax.experimental.pallas import tpu_sc as plsc

NC = 2
NS = 16
LN = 16
NW = NC * NS

_INTERPRET = False


def _mesh():
  return plsc.VectorSubcoreMesh(
      core_axis_name="c", subcore_axis_name="s", num_cores=NC,
      num_subcores=NS)


def _sc_params():
  return pltpu.CompilerParams(
      use_tc_tiling_on_sc=False, needs_layout_passes=False)


def _wid():
  return lax.axis_index("s") * NC + lax.axis_index("c")


def _iota16():
  return lax.iota(jnp.int32, 16)


def _splat(x):
  return jnp.full((16,), x, jnp.int32)


def _rsqrt(x):
  i = plsc.bitcast(x, jnp.int32)
  i = jnp.int32(0x5F3759DF) - lax.shift_right_arithmetic(i, 1)
  y = plsc.bitcast(i, jnp.float32)
  xh = x * jnp.float32(0.5)
  for _ in range(3):
    y = y * (jnp.float32(1.5) - xh * y * y)
  return y


def _sqrt16(x):
  return x * _rsqrt(x)


def _geom_body(n, e, t, q, c_sz,
               px, py, pz, src, dst, ti, tj, tk, qi, qj, qk, ql,
               dist_o, ts_part, qs_part,
               pxv, pyv, pzv, ts_acc, qs_acc, i0, i1, i2, i3, fb):
  w = _wid()
  it = _iota16()
  lane_masks = [it == j for j in range(16)]
  per = e // NW
  nch = per // c_sz
  ng = c_sz // 16

  pltpu.sync_copy(px, pxv)
  pltpu.sync_copy(py, pyv)
  pltpu.sync_copy(pz, pzv)

  def zero(ref, cnt):
    def zb(i, _):
      ref[pl.ds(i * 16, 16)] = jnp.zeros((16,), jnp.float32)
      return 0
    lax.fori_loop(0, cnt, zb, 0)

  zero(ts_acc, n // 16)
  zero(qs_acc, n // 16)

  def echunk(ci, _):
    base = w * per + ci * c_sz
    pltpu.sync_copy(src.at[pl.ds(base, c_sz)], i0)
    pltpu.sync_copy(dst.at[pl.ds(base, c_sz)], i1)

    for g in range(ng):
      s = i0[pl.ds(g * 16, 16)]
      d = i1[pl.ds(g * 16, 16)]
      dx = plsc.load_gather(pxv, [s]) - plsc.load_gather(pxv, [d])
      dy = plsc.load_gather(pyv, [s]) - plsc.load_gather(pyv, [d])
      dz = plsc.load_gather(pzv, [s]) - plsc.load_gather(pzv, [d])
      ss = dx * dx + dy * dy + dz * dz + jnp.float32(1e-8)
      fb[pl.ds(g * 16, 16)] = _sqrt16(ss)
    pltpu.sync_copy(fb, dist_o.at[pl.ds(base, c_sz)])
    return 0
  lax.fori_loop(0, nch, echunk, 0)

  def tchunk(ci, _):
    base = w * per + ci * c_sz
    pltpu.sync_copy(ti.at[pl.ds(base, c_sz)], i0)
    pltpu.sync_copy(tj.at[pl.ds(base, c_sz)], i1)
    pltpu.sync_copy(tk.at[pl.ds(base, c_sz)], i2)

    for g in range(ng):
      a = i0[pl.ds(g * 16, 16)]
      b = i1[pl.ds(g * 16, 16)]
      c = i2[pl.ds(g * 16, 16)]
      bx = plsc.load_gather(pxv, [b])
      by = plsc.load_gather(pyv, [b])
      bz = plsc.load_gather(pzv, [b])
      v1x = plsc.load_gather(pxv, [a]) - bx
      v1y = plsc.load_gather(pyv, [a]) - by
      v1z = plsc.load_gather(pzv, [a]) - bz
      v2x = plsc.load_gather(pxv, [c]) - bx
      v2y = plsc.load_gather(pyv, [c]) - by
      v2z = plsc.load_gather(pzv, [c]) - bz
      dot = v1x * v2x + v1y * v2y + v1z * v2z
      s1 = v1x * v1x + v1y * v1y + v1z * v1z
      s2 = v2x * v2x + v2y * v2y + v2z * v2z
      cos = dot / (_sqrt16(s1 * s2) + jnp.float32(1e-8))
      for lm in lane_masks:
        plsc.addupdate_scatter(ts_acc, [b], cos, mask=lm)
    return 0
  lax.fori_loop(0, nch, tchunk, 0)

  def qchunk(ci, _):
    base = w * per + ci * c_sz
    pltpu.sync_copy(qi.at[pl.ds(base, c_sz)], i0)
    pltpu.sync_copy(qj.at[pl.ds(base, c_sz)], i1)
    pltpu.sync_copy(qk.at[pl.ds(base, c_sz)], i2)
    pltpu.sync_copy(ql.at[pl.ds(base, c_sz)], i3)

    for g in range(ng):
      a = i0[pl.ds(g * 16, 16)]
      b = i1[pl.ds(g * 16, 16)]
      c = i2[pl.ds(g * 16, 16)]
      d = i3[pl.ds(g * 16, 16)]
      ax = plsc.load_gather(pxv, [a]); ay = plsc.load_gather(pyv, [a]); az = plsc.load_gather(pzv, [a])
      bx = plsc.load_gather(pxv, [b]); by = plsc.load_gather(pyv, [b]); bz = plsc.load_gather(pzv, [b])
      cx = plsc.load_gather(pxv, [c]); cy = plsc.load_gather(pyv, [c]); cz = plsc.load_gather(pzv, [c])
      ex = plsc.load_gather(pxv, [d]); ey = plsc.load_gather(pyv, [d]); ez = plsc.load_gather(pzv, [d])
      b1x = bx - ax; b1y = by - ay; b1z = bz - az
      b2x = cx - bx; b2y = cy - by; b2z = cz - bz
      b3x = ex - cx; b3y = ey - cy; b3z = ez - cz
      n1x = b1y * b2z - b1z * b2y
      n1y = b1z * b2x - b1x * b2z
      n1z = b1x * b2y - b1y * b2x
      n2x = b2y * b3z - b2z * b3y
      n2y = b2z * b3x - b2x * b3z
      n2z = b2x * b3y - b2y * b3x
      dot = n1x * n2x + n1y * n2y + n1z * n2z
      s1 = n1x * n1x + n1y * n1y + n1z * n1z
      s2 = n2x * n2x + n2y * n2y + n2z * n2z
      cos = dot / (_sqrt16(s1 * s2) + jnp.float32(1e-8))
      for lm in lane_masks:
        plsc.addupdate_scatter(qs_acc, [b], cos, mask=lm)
    return 0
  lax.fori_loop(0, nch, qchunk, 0)

  pltpu.sync_copy(ts_acc, ts_part.at[w])
  pltpu.sync_copy(qs_acc, qs_part.at[w])


def _geom_call(pos3, src, dst, tix, tjx, tkx, qix, qjx, qkx, qlx):
  n = pos3.shape[0] // 3
  e = src.shape[0]
  c_sz = 80 if e % (NW * 80) == 0 else 16
  px = pos3[:n]
  py = pos3[n:2 * n]
  pz = pos3[2 * n:]
  body = functools.partial(_geom_body, n, e, tix.shape[0], qix.shape[0],
                           c_sz)
  f = pl.kernel(
      body,
      out_type=(
          jax.ShapeDtypeStruct((e,), jnp.float32),
          jax.ShapeDtypeStruct((NW, n), jnp.float32),
          jax.ShapeDtypeStruct((NW, n), jnp.float32),
      ),
      mesh=_mesh(),
      compiler_params=_sc_params(),
      scratch_types=(
          pltpu.VMEM((n,), jnp.float32),
          pltpu.VMEM((n,), jnp.float32),
          pltpu.VMEM((n,), jnp.float32),
          pltpu.VMEM((n,), jnp.float32),
          pltpu.VMEM((n,), jnp.float32),
          pltpu.VMEM((c_sz,), jnp.int32),
          pltpu.VMEM((c_sz,), jnp.int32),
          pltpu.VMEM((c_sz,), jnp.int32),
          pltpu.VMEM((c_sz,), jnp.int32),
          pltpu.VMEM((c_sz,), jnp.float32),
      ),
      interpret=_INTERPRET,
      name="sc_geom",
  )
  return f(px, py, pz, src, dst, tix, tjx, tkx, qix, qjx, qkx, qlx)


def _score_body(n, e, c_sz,
                stab, base_l, src, dst,
                ex_o, den_part,
                srows, drows, bbuf, exb, sidx, didx, den_acc, sem):
  w = _wid()
  it = _iota16()
  lo8 = it < 8
  hi8 = it >= 8
  pair = lax.shift_right_logical(it, 3)
  h8 = jnp.bitwise_and(it, 7)
  per = e // NW
  nch = per // c_sz

  def zero(i, _):
    den_acc[pl.ds(i * 16, 16)] = jnp.zeros((16,), jnp.float32)
    return 0
  lax.fori_loop(0, n * 8 // 16, zero, 0)

  h8p8 = h8 + 8

  def chunk(ci, _):
    base = w * per + ci * c_sz
    pltpu.sync_copy(src.at[pl.ds(base, c_sz)], sidx)
    pltpu.sync_copy(dst.at[pl.ds(base, c_sz)], didx)
    pltpu.async_copy(stab.at[sidx], srows, sem).wait()
    pltpu.async_copy(stab.at[didx], drows, sem).wait()
    pltpu.sync_copy(base_l.at[pl.ds(base, c_sz)], bbuf)

    for g in range(c_sz // 2):
      rp = pair + 2 * g
      sv = plsc.load_gather(srows, [rp, h8])
      dv = plsc.load_gather(drows, [rp, h8p8])
      bv = plsc.load_gather(bbuf, [rp, h8])
      sc = sv + dv + bv
      sc = jnp.maximum(sc, sc * jnp.float32(0.2))
      ev = jnp.exp(sc)
      plsc.store_scatter(exb, [rp, h8], ev)
      dg = plsc.load_gather(didx, [rp])
      tgt = dg * 8 + h8
      plsc.addupdate_scatter(den_acc, [tgt], ev, mask=lo8)
      plsc.addupdate_scatter(den_acc, [tgt], ev, mask=hi8)
    pltpu.sync_copy(exb, ex_o.at[pl.ds(base, c_sz)])
    return 0
  lax.fori_loop(0, nch, chunk, 0)
  pltpu.sync_copy(den_acc, den_part.at[w])


def _score_call(stab, base_l, src, dst):
  n = stab.shape[0]
  e = src.shape[0]
  c_sz = 80 if e % (NW * 80) == 0 else 16
  body = functools.partial(_score_body, n, e, c_sz)
  f = pl.kernel(
      body,
      out_type=(
          jax.ShapeDtypeStruct((e, 8), jnp.float32),
          jax.ShapeDtypeStruct((NW, n * 8), jnp.float32),
      ),
      mesh=_mesh(),
      compiler_params=_sc_params(),
      scratch_types=(
          pltpu.VMEM((c_sz, 16), jnp.float32),
          pltpu.VMEM((c_sz, 16), jnp.float32),
          pltpu.VMEM((c_sz, 8), jnp.float32),
          pltpu.VMEM((c_sz, 8), jnp.float32),
          pltpu.VMEM((c_sz,), jnp.int32),
          pltpu.VMEM((c_sz,), jnp.int32),
          pltpu.VMEM((n * 8,), jnp.float32),
          pltpu.SemaphoreType.DMA,
      ),
      interpret=_INTERPRET,
      name="sc_score",
  )
  return f(stab, base_l, src, dst)


def _agg_body(n, e, c_sz,
              z2, ex, src, dst,
              num_o,
              zrows, exb, sidx, didx, zidx, zzero, acc, sem):
  c = lax.axis_index("c")
  s = lax.axis_index("s")
  it = _iota16()
  per = e // NS
  nch = per // c_sz
  rows_t = n // NS
  zch = 125 if rows_t % 125 == 0 else rows_t
  nzch = rows_t // zch

  zv = jnp.zeros((16,), jnp.float32)
  def zb(i, _):
    for j in range(8):
      plsc.store_scatter(zzero, [_splat(i), it + j * 16], zv)
    return 0
  lax.fori_loop(0, zch, zb, 0)
  def zdma(m, _):
    pltpu.sync_copy(zzero, acc.at[pl.ds(s * rows_t + m * zch, zch)])
    return 0
  lax.fori_loop(0, nzch, zdma, 0)
  plsc.subcore_barrier()

  col_splats = [_splat(0) + c * 4 + jh for jh in range(4)]

  def chunk(ci, _):
    base = s * per + ci * c_sz
    pltpu.sync_copy(src.at[pl.ds(base, c_sz)], sidx)
    pltpu.sync_copy(dst.at[pl.ds(base, c_sz)], didx)
    for k in range(c_sz // 16):
      zidx[pl.ds(k * 16, 16)] = sidx[pl.ds(k * 16, 16)] + c * n
    pltpu.async_copy(z2.at[zidx], zrows, sem).wait()
    pltpu.sync_copy(ex.at[pl.ds(base, c_sz)], exb)

    for k in range(c_sz):
      ks = _splat(k)
      for jh in range(4):
        hv = plsc.load_gather(exb, [ks, col_splats[jh]])
        for jj in range(2):
          j = jh * 2 + jj
          zrows[k, pl.ds(j * 16, 16)] = zrows[k, pl.ds(j * 16, 16)] * hv
    pltpu.sync_copy(zrows, acc.at[didx], add=True)
    return 0
  lax.fori_loop(0, nch, chunk, 0)
  plsc.subcore_barrier()

  def dump(m, _):
    off = s * rows_t + m * zch
    pltpu.sync_copy(acc.at[pl.ds(off, zch)],
                    num_o.at[pl.ds(c * n + off, zch)])
    return 0
  lax.fori_loop(0, nzch, dump, 0)


def _agg_call(z2, ex, src, dst):
  n = z2.shape[0] // 2
  e = src.shape[0]
  c_sz = 80 if e % (NS * 80) == 0 else 16
  rows_t = n // NS
  zch = 125 if rows_t % 125 == 0 else rows_t
  body = functools.partial(_agg_body, n, e, c_sz)
  f = pl.kernel(
      body,
      out_type=jax.ShapeDtypeStruct((2 * n, 128), jnp.float32),
      mesh=_mesh(),
      compiler_params=_sc_params(),
      scratch_types=(
          pltpu.VMEM((c_sz, 128), jnp.float32),
          pltpu.VMEM((c_sz, 8), jnp.float32),
          pltpu.VMEM((c_sz,), jnp.int32),
          pltpu.VMEM((c_sz,), jnp.int32),
          pltpu.VMEM((c_sz,), jnp.int32),
          pltpu.VMEM((zch, 128), jnp.float32),
          pltpu.VMEM_SHARED((n, 128), jnp.float32),
          pltpu.SemaphoreType.DMA,
      ),
      interpret=_INTERPRET,
      name="sc_agg",
  )
  return f(z2, ex, src, dst)


def _maxpool_body(n, g_cnt,
                  h2, batch,
                  maxpart,
                  acc, rows, bb, sem):
  c = lax.axis_index("c")
  s = lax.axis_index("s")
  w = _wid()
  it = _iota16()
  lane_masks = [it == j for j in range(16)]
  groups = n // 16
  gper = groups // NS
  rem = groups - gper * NS
  lo = s * gper + jnp.minimum(s, rem)
  cnt = gper + jnp.where(s < rem, 1, 0)

  ninf = jnp.full((16,), -1e30, jnp.float32)
  def ib(i, _):
    for j in range(8):
      plsc.store_scatter(acc, [_splat(i), it + j * 16], ninf)
    return 0
  lax.fori_loop(0, g_cnt, ib, 0)

  def grp(gg, _):
    pltpu.sync_copy(h2.at[pl.ds(c * n + gg * 16, 16)], rows)
    pltpu.sync_copy(batch.at[pl.ds(gg * 16, 16)], bb)
    bv = bb[:]
    for lane in range(16):
      gid = jnp.max(jnp.where(lane_masks[lane], bv, 0))
      for j in range(8):
        cp = it + j * 16
        cur = plsc.load_gather(acc, [_splat(gid), cp])
        row = plsc.load_gather(rows, [_splat(lane), cp])
        plsc.store_scatter(acc, [_splat(gid), cp], jnp.maximum(cur, row))
    return 0
  lax.fori_loop(lo, lo + cnt, grp, 0)
  pltpu.sync_copy(acc, maxpart.at[w])


def _maxpool_call(h2, batch, g_cnt):
  n = h2.shape[0] // 2
  body = functools.partial(_maxpool_body, n, g_cnt)
  f = pl.kernel(
      body,
      out_type=jax.ShapeDtypeStruct((NW, g_cnt, 128), jnp.float32),
      mesh=_mesh(),
      compiler_params=_sc_params(),
      scratch_types=(
          pltpu.VMEM((g_cnt, 128), jnp.float32),
          pltpu.VMEM((16, 128), jnp.float32),
          pltpu.VMEM((16,), jnp.int32),
          pltpu.SemaphoreType.DMA,
      ),
      interpret=_INTERPRET,
      name="sc_maxpool",
  )
  return f(h2, batch)


def _tc_call(body, grid, in_specs, out_specs, out_shape, name):
  return pl.pallas_call(
      body,
      grid=grid,
      in_specs=in_specs,
      out_specs=out_specs,
      out_shape=out_shape,
      interpret=_INTERPRET,
      name=name,
  )


def _prep_base(edge_attr, dist, we_all, wd_all):
  e = edge_attr.shape[0]
  be = min(e, 2048)
  nl = we_all.shape[1] // 8

  def body(ea_ref, d_ref, we_ref, wd_ref, o_ref):
    v = jnp.dot(ea_ref[...], we_ref[...],
                preferred_element_type=jnp.float32)
    v = v + d_ref[...] * wd_ref[...]
    for l in range(nl):
      o_ref[l] = v[:, 8 * l:8 * l + 8]

  return _tc_call(
      body, (pl.cdiv(e, be),),
      [
          pl.BlockSpec((be, 16), lambda i: (i, 0)),
          pl.BlockSpec((be, 1), lambda i: (i, 0)),
          pl.BlockSpec((16, 8 * nl), lambda i: (0, 0)),
          pl.BlockSpec((1, 8 * nl), lambda i: (0, 0)),
      ],
      pl.BlockSpec((nl, be, 8), lambda i: (0, i, 0)),
      jax.ShapeDtypeStruct((nl, e, 8), jnp.float32),
      "tc_prep_base")(edge_attr, dist, we_all, wd_all)


def _layer_head(h, w_mat, wr_mat, sa_mat, extras=None):
  n, din = h.shape
  bn = min(n, 2048)
  hid = w_mat.shape[1]
  with_parts = extras is not None

  def body(*refs):
    if with_parts:
      h_ref, w_ref, wr_ref, sa_ref, tp_ref, qp_ref, ones_ref, \
          z2_ref, s_ref, r_ref, ts_ref, qs_ref = refs
    else:
      h_ref, w_ref, wr_ref, sa_ref, z2_ref, s_ref, r_ref = refs
    z = jnp.dot(h_ref[...], w_ref[...], preferred_element_type=jnp.float32)
    z2_ref[0] = z[:, :hid // 2]
    z2_ref[1] = z[:, hid // 2:]
    s_ref[...] = jnp.dot(z, sa_ref[...], preferred_element_type=jnp.float32,
                         precision=lax.Precision.HIGHEST)
    r_ref[...] = jnp.dot(h_ref[...], wr_ref[...],
                         preferred_element_type=jnp.float32)
    if with_parts:
      ts_ref[...] = lax.dot_general(
          tp_ref[...], ones_ref[...], (((0,), (0,)), ((), ())),
          preferred_element_type=jnp.float32,
          precision=lax.Precision.HIGHEST)
      qs_ref[...] = lax.dot_general(
          qp_ref[...], ones_ref[...], (((0,), (0,)), ((), ())),
          preferred_element_type=jnp.float32,
          precision=lax.Precision.HIGHEST)

  in_specs = [
      pl.BlockSpec((bn, din), lambda i: (i, 0)),
      pl.BlockSpec((din, hid), lambda i: (0, 0)),
      pl.BlockSpec((din, hid), lambda i: (0, 0)),
      pl.BlockSpec((hid, 16), lambda i: (0, 0)),
  ]
  args = [h, w_mat, wr_mat, sa_mat]
  out_specs = [
      pl.BlockSpec((2, bn, hid // 2), lambda i: (0, i, 0)),
      pl.BlockSpec((bn, 16), lambda i: (i, 0)),
      pl.BlockSpec((bn, hid), lambda i: (i, 0)),
  ]
  out_shape = [
      jax.ShapeDtypeStruct((2, n, hid // 2), jnp.float32),
      jax.ShapeDtypeStruct((n, 16), jnp.float32),
      jax.ShapeDtypeStruct((n, hid), jnp.float32),
  ]
  if with_parts:
    ts_part, qs_part = extras
    nwp = ts_part.shape[0]
    in_specs += [
        pl.BlockSpec((nwp, bn), lambda i: (0, i)),
        pl.BlockSpec((nwp, bn), lambda i: (0, i)),
        pl.BlockSpec((nwp, 1), lambda i: (0, 0)),
    ]
    args += [ts_part, qs_part, jnp.ones((nwp, 1), jnp.float32)]
    out_specs += [
        pl.BlockSpec((bn, 1), lambda i: (i, 0)),
        pl.BlockSpec((bn, 1), lambda i: (i, 0)),
    ]
    out_shape += [
        jax.ShapeDtypeStruct((n, 1), jnp.float32),
        jax.ShapeDtypeStruct((n, 1), jnp.float32),
    ]
  return _tc_call(body, (pl.cdiv(n, bn),), in_specs, out_specs, out_shape,
                  "tc_layer_head")(*args)


def _den_reduce(den_part_flat):
  nwp, m = den_part_flat.shape
  bm = 8192 if m > 8192 else m

  def body(dp_ref, o_ref):
    o_ref[...] = jnp.sum(dp_ref[...], axis=0, keepdims=True)

  return _tc_call(
      body, (pl.cdiv(m, bm),),
      [pl.BlockSpec((nwp, bm), lambda i: (0, i))],
      pl.BlockSpec((1, bm), lambda i: (0, i)),
      jax.ShapeDtypeStruct((1, m), jnp.float32),
      "tc_den_reduce")(den_part_flat)


def _node_update(num2, den, r, ts, qs, wt_l, wq_l, ones_exp):
  n = r.shape[0]
  hid = r.shape[1]
  bn = min(n, 2048)

  def body(num_ref, den_ref, r_ref, ts_ref, qs_ref, wt_ref, wq_ref,
           oe_ref, h_ref):
    num = jnp.concatenate([num_ref[0], num_ref[1]], axis=-1)
    den = den_ref[...] + jnp.float32(1e-9)
    den_e = jnp.dot(den, oe_ref[...], preferred_element_type=jnp.float32,
                    precision=lax.Precision.HIGHEST)
    v = num / den_e + r_ref[...] + ts_ref[...] * wt_ref[...] \
        + qs_ref[...] * wq_ref[...]
    h_ref[...] = jnp.where(v > 0, v, jnp.exp(jnp.minimum(v, 0.0)) - 1.0)

  return _tc_call(
      body, (pl.cdiv(n, bn),),
      [
          pl.BlockSpec((2, bn, hid // 2), lambda i: (0, i, 0)),
          pl.BlockSpec((bn, 8), lambda i: (i, 0)),
          pl.BlockSpec((bn, hid), lambda i: (i, 0)),
          pl.BlockSpec((bn, 1), lambda i: (i, 0)),
          pl.BlockSpec((bn, 1), lambda i: (i, 0)),
          pl.BlockSpec((1, hid), lambda i: (0, 0)),
          pl.BlockSpec((1, hid), lambda i: (0, 0)),
          pl.BlockSpec((8, hid), lambda i: (0, 0)),
      ],
      pl.BlockSpec((bn, hid), lambda i: (i, 0)),
      jax.ShapeDtypeStruct((n, hid), jnp.float32),
      "tc_node_update")(num2, den, r, ts, qs, wt_l, wq_l, ones_exp)


def _split_rows(h):
  n, hid = h.shape
  bn = min(n, 2048)

  def body(h_ref, o_ref):
    o_ref[0] = h_ref[:, :hid // 2]
    o_ref[1] = h_ref[:, hid // 2:]

  return _tc_call(
      body, (pl.cdiv(n, bn),),
      [pl.BlockSpec((bn, hid), lambda i: (i, 0))],
      pl.BlockSpec((2, bn, hid // 2), lambda i: (0, i, 0)),
      jax.ShapeDtypeStruct((2, n, hid // 2), jnp.float32),
      "tc_split")(h)


def _sum_pool(h, batch_col, g_cnt):
  n, hid = h.shape
  bn = min(n, 2048)

  def body(h_ref, b_ref, o_ref):
    @pl.when(pl.program_id(0) == 0)
    def _():
      o_ref[...] = jnp.zeros_like(o_ref)
    valid = n - pl.program_id(0) * bn
    rows = lax.broadcasted_iota(jnp.int32, (bn, g_cnt), 0)
    onehot = (b_ref[...] == lax.broadcasted_iota(jnp.int32, (bn, g_cnt), 1)
              ).astype(jnp.float32)
    onehot = jnp.where(rows < valid, onehot, 0.0)
    o_ref[...] += lax.dot_general(
        onehot, h_ref[...], (((0,), (0,)), ((), ())),
        preferred_element_type=jnp.float32,
        precision=lax.Precision.HIGHEST)

  return _tc_call(
      body, (pl.cdiv(n, bn),),
      [
          pl.BlockSpec((bn, hid), lambda i: (i, 0)),
          pl.BlockSpec((bn, 1), lambda i: (i, 0)),
      ],
      pl.BlockSpec((g_cnt, hid), lambda i: (0, 0)),
      jax.ShapeDtypeStruct((g_cnt, hid), jnp.float32),
      "tc_sum_pool")(h, batch_col)


def _readout(sum_pool, maxpart, temps, m1a, m1b, m1c, b1, m2, b2, m3, b3):
  g_cnt = sum_pool.shape[0]
  nwp = maxpart.shape[0]

  def body(sp_ref, mp_ref, t_ref, m1a_ref, m1b_ref, m1c_ref, b1_ref,
           m2_ref, b2_ref, m3_ref, b3_ref, o_ref):
    mp0 = mp_ref[0]
    mp1 = mp_ref[1]
    for k in range(2, nwp, 2):
      mp0 = jnp.maximum(mp0, mp_ref[k])
      mp1 = jnp.maximum(mp1, mp_ref[k + 1])
    mp = jnp.concatenate([mp0, mp1], axis=-1)
    mp = jnp.where(mp <= -1e29, 0.0, mp)
    u = jnp.dot(sp_ref[...], m1a_ref[...], preferred_element_type=jnp.float32)
    u += jnp.dot(mp, m1b_ref[...], preferred_element_type=jnp.float32)
    u += jnp.dot(t_ref[...], m1c_ref[...], preferred_element_type=jnp.float32)
    u += b1_ref[...]
    u = jnp.where(u > 0, u, jnp.exp(jnp.minimum(u, 0.0)) - 1.0)
    u = jnp.dot(u, m2_ref[...], preferred_element_type=jnp.float32) + b2_ref[...]
    u = jnp.where(u > 0, u, jnp.exp(jnp.minimum(u, 0.0)) - 1.0)
    o_ref[...] = jnp.dot(u, m3_ref[...],
                         preferred_element_type=jnp.float32) + b3_ref[...]

  hid = sum_pool.shape[1]
  return _tc_call(
      body, (1,),
      [
          pl.BlockSpec((g_cnt, hid), lambda i: (0, 0)),
          pl.BlockSpec((nwp, g_cnt, hid // 2), lambda i: (0, 0, 0)),
          pl.BlockSpec((g_cnt, 1), lambda i: (0, 0)),
          pl.BlockSpec((hid, 256), lambda i: (0, 0)),
          pl.BlockSpec((hid, 256), lambda i: (0, 0)),
          pl.BlockSpec((1, 256), lambda i: (0, 0)),
          pl.BlockSpec((1, 256), lambda i: (0, 0)),
          pl.BlockSpec((256, 256), lambda i: (0, 0)),
          pl.BlockSpec((1, 256), lambda i: (0, 0)),
          pl.BlockSpec((256, 1), lambda i: (0, 0)),
          pl.BlockSpec((1, 1), lambda i: (0, 0)),
      ],
      pl.BlockSpec((g_cnt, 1), lambda i: (0, 0)),
      jax.ShapeDtypeStruct((g_cnt, 1), jnp.float32),
      "tc_readout")(sum_pool, maxpart, temps, m1a, m1b, m1c, b1, m2, b2,
                    m3, b3)


def _block_diag_heads(a):
  h, dh = a.shape
  eye = jnp.eye(h, dtype=a.dtype)
  return (eye[:, None, :] * a[:, :, None]).reshape(h * dh, h)


def kernel(x, pos, edge_attr, temps, edge_index, triple_index, quadra_index,
           batch, W0, W12, Wres0, Wres12, A_src, A_dst, We, wd, wt, wq,
           M1, b1, M2, b2, M3, b3):
  n = x.shape[0]
  g_cnt = temps.shape[0]
  hid = W0.shape[1]
  i32 = jnp.int32

  src = edge_index[0].astype(i32)
  dst = edge_index[1].astype(i32)
  tix = triple_index[0].astype(i32)
  tjx = triple_index[1].astype(i32)
  tkx = triple_index[2].astype(i32)
  qix = quadra_index[0].astype(i32)
  qjx = quadra_index[1].astype(i32)
  qkx = quadra_index[2].astype(i32)
  qlx = quadra_index[3].astype(i32)
  batch_i = batch.astype(i32)
  pos3 = jnp.concatenate([pos[:, 0], pos[:, 1], pos[:, 2]])

  dist, ts_part, qs_part = _geom_call(pos3, src, dst, tix, tjx, tkx,
                                      qix, qjx, qkx, qlx)

  we_all = jnp.transpose(We, (1, 0, 2)).reshape(16, -1)
  wd_all = wd.reshape(1, -1)
  base = _prep_base(edge_attr, dist.reshape(-1, 1), we_all, wd_all)

  ones_exp = jnp.repeat(jnp.eye(8, dtype=jnp.float32), hid // 8, axis=1)

  ws = [W0, W12[0], W12[1]]
  wrs = [Wres0, Wres12[0], Wres12[1]]
  sa_mats = [jnp.concatenate(
      [_block_diag_heads(A_src[l]), _block_diag_heads(A_dst[l])], axis=1)
      for l in range(3)]

  h = x
  ts = qs = None
  for l in range(3):
    if l == 0:
      z2, stab, r, ts, qs = _layer_head(h, ws[l], wrs[l], sa_mats[l],
                                        extras=(ts_part, qs_part))
    else:
      z2, stab, r = _layer_head(h, ws[l], wrs[l], sa_mats[l])
    ex, den_part = _score_call(stab, base[l], src, dst)
    num2 = _agg_call(z2.reshape(2 * n, hid // 2), ex, src, dst)
    den = _den_reduce(den_part).reshape(n, 8)
    h = _node_update(num2.reshape(2, n, hid // 2), den, r, ts, qs,
                     wt[l].reshape(1, hid), wq[l].reshape(1, hid), ones_exp)

  h2 = _split_rows(h)
  maxpart = _maxpool_call(h2.reshape(2 * n, hid // 2), batch_i, g_cnt)
  sp = _sum_pool(h, batch_i.reshape(n, 1), g_cnt)
  out = _readout(sp, maxpart, temps, M1[:hid], M1[hid:2 * hid],
                 M1[2 * hid:], b1.reshape(1, -1), M2, b2.reshape(1, -1),
                 M3, b3.reshape(1, 1))
  return out


# speedup vs baseline: 23.9092x; 1.2982x over previous
import functools
from typing import Any

import jax
import jax.numpy as jnp
from jax import lax
from jax.experimental import pallas as pl
from jax.experimental.pallas import tpu as pltpu
from jax.experimental.pallas import tpu_sc as plsc

NC = 2
NS = 16
LN = 16
NW = NC * NS

_INTERPRET = False


def _mesh():
  return plsc.VectorSubcoreMesh(
      core_axis_name="c", subcore_axis_name="s", num_cores=NC,
      num_subcores=NS)


def _sc_params():
  return pltpu.CompilerParams(
      use_tc_tiling_on_sc=False, needs_layout_passes=False)


def _wid():
  return lax.axis_index("s") * NC + lax.axis_index("c")


def _iota16():
  return lax.iota(jnp.int32, 16)


def _splat(x):
  return jnp.full((16,), x, jnp.int32)


def _rsqrt(x):
  i = plsc.bitcast(x, jnp.int32)
  i = jnp.int32(0x5F3759DF) - lax.shift_right_arithmetic(i, 1)
  y = plsc.bitcast(i, jnp.float32)
  xh = x * jnp.float32(0.5)
  for _ in range(3):
    y = y * (jnp.float32(1.5) - xh * y * y)
  return y


def _sqrt16(x):
  return x * _rsqrt(x)


def _geom_body(n, e, t, q, c_sz,
               px, py, pz, src, dst, ti, tj, tk, qi, qj, qk, ql,
               dist_o, ts_part, qs_part,
               pxv, pyv, pzv, ts_acc, qs_acc, i0, i1, i2, i3, fb):
  w = _wid()
  it = _iota16()
  lane_masks = [it == j for j in range(16)]
  per = e // NW
  nch = per // c_sz
  ng = c_sz // 16

  pltpu.sync_copy(px, pxv)
  pltpu.sync_copy(py, pyv)
  pltpu.sync_copy(pz, pzv)

  def zero(ref, cnt):
    def zb(i, _):
      ref[pl.ds(i * 16, 16)] = jnp.zeros((16,), jnp.float32)
      return 0
    lax.fori_loop(0, cnt, zb, 0)

  zero(ts_acc, n // 16)
  zero(qs_acc, n // 16)

  def echunk(ci, _):
    base = w * per + ci * c_sz
    pltpu.sync_copy(src.at[pl.ds(base, c_sz)], i0)
    pltpu.sync_copy(dst.at[pl.ds(base, c_sz)], i1)

    for g in range(ng):
      s = i0[pl.ds(g * 16, 16)]
      d = i1[pl.ds(g * 16, 16)]
      dx = plsc.load_gather(pxv, [s]) - plsc.load_gather(pxv, [d])
      dy = plsc.load_gather(pyv, [s]) - plsc.load_gather(pyv, [d])
      dz = plsc.load_gather(pzv, [s]) - plsc.load_gather(pzv, [d])
      ss = dx * dx + dy * dy + dz * dz + jnp.float32(1e-8)
      fb[pl.ds(g * 16, 16)] = _sqrt16(ss)
    pltpu.sync_copy(fb, dist_o.at[pl.ds(base, c_sz)])
    return 0
  lax.fori_loop(0, nch, echunk, 0)

  def tchunk(ci, _):
    base = w * per + ci * c_sz
    pltpu.sync_copy(ti.at[pl.ds(base, c_sz)], i0)
    pltpu.sync_copy(tj.at[pl.ds(base, c_sz)], i1)
    pltpu.sync_copy(tk.at[pl.ds(base, c_sz)], i2)

    for g in range(ng):
      a = i0[pl.ds(g * 16, 16)]
      b = i1[pl.ds(g * 16, 16)]
      c = i2[pl.ds(g * 16, 16)]
      bx = plsc.load_gather(pxv, [b])
      by = plsc.load_gather(pyv, [b])
      bz = plsc.load_gather(pzv, [b])
      v1x = plsc.load_gather(pxv, [a]) - bx
      v1y = plsc.load_gather(pyv, [a]) - by
      v1z = plsc.load_gather(pzv, [a]) - bz
      v2x = plsc.load_gather(pxv, [c]) - bx
      v2y = plsc.load_gather(pyv, [c]) - by
      v2z = plsc.load_gather(pzv, [c]) - bz
      dot = v1x * v2x + v1y * v2y + v1z * v2z
      s1 = v1x * v1x + v1y * v1y + v1z * v1z
      s2 = v2x * v2x + v2y * v2y + v2z * v2z
      cos = dot / (_sqrt16(s1 * s2) + jnp.float32(1e-8))
      for lm in lane_masks:
        plsc.addupdate_scatter(ts_acc, [b], cos, mask=lm)
    return 0
  lax.fori_loop(0, nch, tchunk, 0)

  def qchunk(ci, _):
    base = w * per + ci * c_sz
    pltpu.sync_copy(qi.at[pl.ds(base, c_sz)], i0)
    pltpu.sync_copy(qj.at[pl.ds(base, c_sz)], i1)
    pltpu.sync_copy(qk.at[pl.ds(base, c_sz)], i2)
    pltpu.sync_copy(ql.at[pl.ds(base, c_sz)], i3)

    for g in range(ng):
      a = i0[pl.ds(g * 16, 16)]
      b = i1[pl.ds(g * 16, 16)]
      c = i2[pl.ds(g * 16, 16)]
      d = i3[pl.ds(g * 16, 16)]
      ax = plsc.load_gather(pxv, [a]); ay = plsc.load_gather(pyv, [a]); az = plsc.load_gather(pzv, [a])
      bx = plsc.load_gather(pxv, [b]); by = plsc.load_gather(pyv, [b]); bz = plsc.load_gather(pzv, [b])
      cx = plsc.load_gather(pxv, [c]); cy = plsc.load_gather(pyv, [c]); cz = plsc.load_gather(pzv, [c])
      ex = plsc.load_gather(pxv, [d]); ey = plsc.load_gather(pyv, [d]); ez = plsc.load_gather(pzv, [d])
      b1x = bx - ax; b1y = by - ay; b1z = bz - az
      b2x = cx - bx; b2y = cy - by; b2z = cz - bz
      b3x = ex - cx; b3y = ey - cy; b3z = ez - cz
      n1x = b1y * b2z - b1z * b2y
      n1y = b1z * b2x - b1x * b2z
      n1z = b1x * b2y - b1y * b2x
      n2x = b2y * b3z - b2z * b3y
      n2y = b2z * b3x - b2x * b3z
      n2z = b2x * b3y - b2y * b3x
      dot = n1x * n2x + n1y * n2y + n1z * n2z
      s1 = n1x * n1x + n1y * n1y + n1z * n1z
      s2 = n2x * n2x + n2y * n2y + n2z * n2z
      cos = dot / (_sqrt16(s1 * s2) + jnp.float32(1e-8))
      for lm in lane_masks:
        plsc.addupdate_scatter(qs_acc, [b], cos, mask=lm)
    return 0
  lax.fori_loop(0, nch, qchunk, 0)

  pltpu.sync_copy(ts_acc, ts_part.at[w])
  pltpu.sync_copy(qs_acc, qs_part.at[w])


def _geom_call(pos3, src, dst, tix, tjx, tkx, qix, qjx, qkx, qlx):
  n = pos3.shape[0] // 3
  e = src.shape[0]
  c_sz = 80 if e % (NW * 80) == 0 else 16
  px = pos3[:n]
  py = pos3[n:2 * n]
  pz = pos3[2 * n:]
  body = functools.partial(_geom_body, n, e, tix.shape[0], qix.shape[0],
                           c_sz)
  f = pl.kernel(
      body,
      out_type=(
          jax.ShapeDtypeStruct((e,), jnp.float32),
          jax.ShapeDtypeStruct((NW, n), jnp.float32),
          jax.ShapeDtypeStruct((NW, n), jnp.float32),
      ),
      mesh=_mesh(),
      compiler_params=_sc_params(),
      scratch_types=(
          pltpu.VMEM((n,), jnp.float32),
          pltpu.VMEM((n,), jnp.float32),
          pltpu.VMEM((n,), jnp.float32),
          pltpu.VMEM((n,), jnp.float32),
          pltpu.VMEM((n,), jnp.float32),
          pltpu.VMEM((c_sz,), jnp.int32),
          pltpu.VMEM((c_sz,), jnp.int32),
          pltpu.VMEM((c_sz,), jnp.int32),
          pltpu.VMEM((c_sz,), jnp.int32),
          pltpu.VMEM((c_sz,), jnp.float32),
      ),
      interpret=_INTERPRET,
      name="sc_geom",
  )
  return f(px, py, pz, src, dst, tix, tjx, tkx, qix, qjx, qkx, qlx)


def _score_body(n, e, c_sz,
                stab, base_l, src, dst,
                ex_o, den_part,
                srows, drows, bbuf, exb, sidx, didx, den_acc, sem):
  w = _wid()
  it = _iota16()
  lo8 = it < 8
  hi8 = it >= 8
  pair = lax.shift_right_logical(it, 3)
  h8 = jnp.bitwise_and(it, 7)
  per = e // NW
  nch = per // c_sz

  def zero(i, _):
    den_acc[pl.ds(i * 16, 16)] = jnp.zeros((16,), jnp.float32)
    return 0
  lax.fori_loop(0, n * 8 // 16, zero, 0)

  h8p8 = h8 + 8

  def chunk(ci, _):
    base = w * per + ci * c_sz
    pltpu.sync_copy(src.at[pl.ds(base, c_sz)], sidx)
    pltpu.sync_copy(dst.at[pl.ds(base, c_sz)], didx)
    pltpu.async_copy(stab.at[sidx], srows, sem).wait()
    pltpu.async_copy(stab.at[didx], drows, sem).wait()
    pltpu.sync_copy(base_l.at[pl.ds(base, c_sz)], bbuf)

    for g in range(c_sz // 2):
      rp = pair + 2 * g
      sv = plsc.load_gather(srows, [rp, h8])
      dv = plsc.load_gather(drows, [rp, h8p8])
      bv = plsc.load_gather(bbuf, [rp, h8])
      sc = sv + dv + bv
      sc = jnp.maximum(sc, sc * jnp.float32(0.2))
      ev = jnp.exp(sc)
      plsc.store_scatter(exb, [rp, h8], ev)
      dg = plsc.load_gather(didx, [rp])
      tgt = dg * 8 + h8
      plsc.addupdate_scatter(den_acc, [tgt], ev, mask=lo8)
      plsc.addupdate_scatter(den_acc, [tgt], ev, mask=hi8)
    pltpu.sync_copy(exb, ex_o.at[pl.ds(base, c_sz)])
    return 0
  lax.fori_loop(0, nch, chunk, 0)
  pltpu.sync_copy(den_acc, den_part.at[w])


def _score_call(stab, base_l, src, dst):
  n = stab.shape[0]
  e = src.shape[0]
  c_sz = 80 if e % (NW * 80) == 0 else 16
  body = functools.partial(_score_body, n, e, c_sz)
  f = pl.kernel(
      body,
      out_type=(
          jax.ShapeDtypeStruct((e, 8), jnp.float32),
          jax.ShapeDtypeStruct((NW, n * 8), jnp.float32),
      ),
      mesh=_mesh(),
      compiler_params=_sc_params(),
      scratch_types=(
          pltpu.VMEM((c_sz, 16), jnp.float32),
          pltpu.VMEM((c_sz, 16), jnp.float32),
          pltpu.VMEM((c_sz, 8), jnp.float32),
          pltpu.VMEM((c_sz, 8), jnp.float32),
          pltpu.VMEM((c_sz,), jnp.int32),
          pltpu.VMEM((c_sz,), jnp.int32),
          pltpu.VMEM((n * 8,), jnp.float32),
          pltpu.SemaphoreType.DMA,
      ),
      interpret=_INTERPRET,
      name="sc_score",
  )
  return f(stab, base_l, src, dst)


def _agg_body(n, e, c_sz, sub,
              z2, ex, src2, dst2,
              num_o,
              zrows, exb, didx2, zidx2, zzero, acc, gsem, ssem):
  c = lax.axis_index("c")
  s = lax.axis_index("s")
  per = e // NS
  nsub = per // sub
  spc = c_sz // sub
  nch = per // c_sz
  rows_t = n // NS
  zch = zzero.shape[0]
  nzch = rows_t // zch
  it = _iota16()

  zv = jnp.zeros((16,), jnp.float32)
  def zb(i, _):
    for j in range(8):
      plsc.store_scatter(zzero, [_splat(i), it + j * 16], zv)
    return 0
  lax.fori_loop(0, zch, zb, 0)
  def zdma(m, _):
    pltpu.sync_copy(zzero, acc.at[pl.ds(s * rows_t + m * zch, zch)])
    return 0
  lax.fori_loop(0, nzch, zdma, 0)
  plsc.subcore_barrier()

  col_splats = [_splat(0) + c * 4 + jh for jh in range(4)]

  def chunk(ci, _):
    base = s * per + ci * c_sz
    r0 = s * nsub + ci * spc
    pltpu.sync_copy(src2.at[pl.ds(r0, spc)], zidx2)
    pltpu.sync_copy(dst2.at[pl.ds(r0, spc)], didx2)
    for j in range(spc):
      for k in range(sub // 16):
        zidx2[j, pl.ds(k * 16, 16)] = zidx2[j, pl.ds(k * 16, 16)] + c * n
    descs = [
        pltpu.async_copy(z2.at[zidx2.at[j]],
                         zrows.at[pl.ds(j * sub, sub)], gsem)
        for j in range(spc)
    ]
    pltpu.sync_copy(ex.at[pl.ds(base * 8, c_sz * 8)], exb)
    for d in descs:
      d.wait()

    def edge(k, _):
      ks = _splat(k * 8)
      for jh in range(4):
        hv = plsc.load_gather(exb, [ks + col_splats[jh]])
        for jj in range(2):
          j = jh * 2 + jj
          zrows[k, pl.ds(j * 16, 16)] = zrows[k, pl.ds(j * 16, 16)] * hv
      return 0
    lax.fori_loop(0, c_sz, edge, 0)

    sdescs = [
        pltpu.async_copy(zrows.at[pl.ds(j * sub, sub)],
                         acc.at[didx2.at[j]], ssem, add=True)
        for j in range(spc)
    ]
    for d in sdescs:
      d.wait()
    return 0
  lax.fori_loop(0, nch, chunk, 0)
  plsc.subcore_barrier()

  def dump(m, _):
    off = s * rows_t + m * zch
    pltpu.sync_copy(acc.at[pl.ds(off, zch)],
                    num_o.at[pl.ds(c * n + off, zch)])
    return 0
  lax.fori_loop(0, nzch, dump, 0)


def _agg_call(z2, ex, src, dst):
  n = z2.shape[0] // 2
  e = src.shape[0]
  sub = 80 if e % (NS * 80) == 0 else 16
  c_sz = 2 * sub if e % (NS * 2 * sub) == 0 else sub
  rows_t = n // NS
  zch = 25 if rows_t % 25 == 0 else rows_t
  nsub = e // NS // sub
  src2 = src.reshape(NS * nsub, sub)
  dst2 = dst.reshape(NS * nsub, sub)
  body = functools.partial(_agg_body, n, e, c_sz, sub)
  f = pl.kernel(
      body,
      out_type=jax.ShapeDtypeStruct((2 * n, 128), jnp.float32),
      mesh=_mesh(),
      compiler_params=_sc_params(),
      scratch_types=(
          pltpu.VMEM((c_sz, 128), jnp.float32),
          pltpu.VMEM((c_sz * 8,), jnp.float32),
          pltpu.VMEM((c_sz // sub, sub), jnp.int32),
          pltpu.VMEM((c_sz // sub, sub), jnp.int32),
          pltpu.VMEM((zch, 128), jnp.float32),
          pltpu.VMEM_SHARED((n, 128), jnp.float32),
          pltpu.SemaphoreType.DMA,
          pltpu.SemaphoreType.DMA,
      ),
      interpret=_INTERPRET,
      name="sc_agg",
  )
  return f(z2, ex.reshape(-1), src2, dst2)


def _maxpool_body(n, g_cnt,
                  h2, batch,
                  maxpart,
                  acc, rows, bb, sem):
  c = lax.axis_index("c")
  s = lax.axis_index("s")
  w = _wid()
  it = _iota16()
  lane_masks = [it == j for j in range(16)]
  groups = n // 16
  gper = groups // NS
  rem = groups - gper * NS
  lo = s * gper + jnp.minimum(s, rem)
  cnt = gper + jnp.where(s < rem, 1, 0)

  ninf = jnp.full((16,), -1e30, jnp.float32)
  def ib(i, _):
    for j in range(8):
      plsc.store_scatter(acc, [_splat(i), it + j * 16], ninf)
    return 0
  lax.fori_loop(0, g_cnt, ib, 0)

  def grp(gg, _):
    pltpu.sync_copy(h2.at[pl.ds(c * n + gg * 16, 16)], rows)
    pltpu.sync_copy(batch.at[pl.ds(gg * 16, 16)], bb)
    bv = bb[:]
    for lane in range(16):
      gid = jnp.max(jnp.where(lane_masks[lane], bv, 0))
      for j in range(8):
        cp = it + j * 16
        cur = plsc.load_gather(acc, [_splat(gid), cp])
        row = plsc.load_gather(rows, [_splat(lane), cp])
        plsc.store_scatter(acc, [_splat(gid), cp], jnp.maximum(cur, row))
    return 0
  lax.fori_loop(lo, lo + cnt, grp, 0)
  pltpu.sync_copy(acc, maxpart.at[w])


def _maxpool_call(h2, batch, g_cnt):
  n = h2.shape[0] // 2
  body = functools.partial(_maxpool_body, n, g_cnt)
  f = pl.kernel(
      body,
      out_type=jax.ShapeDtypeStruct((NW, g_cnt, 128), jnp.float32),
      mesh=_mesh(),
      compiler_params=_sc_params(),
      scratch_types=(
          pltpu.VMEM((g_cnt, 128), jnp.float32),
          pltpu.VMEM((16, 128), jnp.float32),
          pltpu.VMEM((16,), jnp.int32),
          pltpu.SemaphoreType.DMA,
      ),
      interpret=_INTERPRET,
      name="sc_maxpool",
  )
  return f(h2, batch)


def _tc_call(body, grid, in_specs, out_specs, out_shape, name):
  return pl.pallas_call(
      body,
      grid=grid,
      in_specs=in_specs,
      out_specs=out_specs,
      out_shape=out_shape,
      interpret=_INTERPRET,
      name=name,
  )


def _prep_base(edge_attr, dist, we_all, wd_all):
  e = edge_attr.shape[0]
  be = min(e, 2048)
  nl = we_all.shape[1] // 8

  def body(ea_ref, d_ref, we_ref, wd_ref, o_ref):
    v = jnp.dot(ea_ref[...], we_ref[...],
                preferred_element_type=jnp.float32)
    v = v + d_ref[...] * wd_ref[...]
    for l in range(nl):
      o_ref[l] = v[:, 8 * l:8 * l + 8]

  return _tc_call(
      body, (pl.cdiv(e, be),),
      [
          pl.BlockSpec((be, 16), lambda i: (i, 0)),
          pl.BlockSpec((be, 1), lambda i: (i, 0)),
          pl.BlockSpec((16, 8 * nl), lambda i: (0, 0)),
          pl.BlockSpec((1, 8 * nl), lambda i: (0, 0)),
      ],
      pl.BlockSpec((nl, be, 8), lambda i: (0, i, 0)),
      jax.ShapeDtypeStruct((nl, e, 8), jnp.float32),
      "tc_prep_base")(edge_attr, dist, we_all, wd_all)


def _layer_head(h, w_mat, wr_mat, sa_mat, extras=None):
  n, din = h.shape
  bn = min(n, 2048)
  hid = w_mat.shape[1]
  with_parts = extras is not None

  def body(*refs):
    if with_parts:
      h_ref, w_ref, wr_ref, sa_ref, tp_ref, qp_ref, ones_ref, \
          z2_ref, s_ref, r_ref, ts_ref, qs_ref = refs
    else:
      h_ref, w_ref, wr_ref, sa_ref, z2_ref, s_ref, r_ref = refs
    z = jnp.dot(h_ref[...], w_ref[...], preferred_element_type=jnp.float32)
    z2_ref[0] = z[:, :hid // 2]
    z2_ref[1] = z[:, hid // 2:]
    s_ref[...] = jnp.dot(z, sa_ref[...], preferred_element_type=jnp.float32,
                         precision=lax.Precision.HIGHEST)
    r_ref[...] = jnp.dot(h_ref[...], wr_ref[...],
                         preferred_element_type=jnp.float32)
    if with_parts:
      ts_ref[...] = lax.dot_general(
          tp_ref[...], ones_ref[...], (((0,), (0,)), ((), ())),
          preferred_element_type=jnp.float32,
          precision=lax.Precision.HIGHEST)
      qs_ref[...] = lax.dot_general(
          qp_ref[...], ones_ref[...], (((0,), (0,)), ((), ())),
          preferred_element_type=jnp.float32,
          precision=lax.Precision.HIGHEST)

  in_specs = [
      pl.BlockSpec((bn, din), lambda i: (i, 0)),
      pl.BlockSpec((din, hid), lambda i: (0, 0)),
      pl.BlockSpec((din, hid), lambda i: (0, 0)),
      pl.BlockSpec((hid, 16), lambda i: (0, 0)),
  ]
  args = [h, w_mat, wr_mat, sa_mat]
  out_specs = [
      pl.BlockSpec((2, bn, hid // 2), lambda i: (0, i, 0)),
      pl.BlockSpec((bn, 16), lambda i: (i, 0)),
      pl.BlockSpec((bn, hid), lambda i: (i, 0)),
  ]
  out_shape = [
      jax.ShapeDtypeStruct((2, n, hid // 2), jnp.float32),
      jax.ShapeDtypeStruct((n, 16), jnp.float32),
      jax.ShapeDtypeStruct((n, hid), jnp.float32),
  ]
  if with_parts:
    ts_part, qs_part = extras
    nwp = ts_part.shape[0]
    in_specs += [
        pl.BlockSpec((nwp, bn), lambda i: (0, i)),
        pl.BlockSpec((nwp, bn), lambda i: (0, i)),
        pl.BlockSpec((nwp, 1), lambda i: (0, 0)),
    ]
    args += [ts_part, qs_part, jnp.ones((nwp, 1), jnp.float32)]
    out_specs += [
        pl.BlockSpec((bn, 1), lambda i: (i, 0)),
        pl.BlockSpec((bn, 1), lambda i: (i, 0)),
    ]
    out_shape += [
        jax.ShapeDtypeStruct((n, 1), jnp.float32),
        jax.ShapeDtypeStruct((n, 1), jnp.float32),
    ]
  return _tc_call(body, (pl.cdiv(n, bn),), in_specs, out_specs, out_shape,
                  "tc_layer_head")(*args)


def _den_reduce(den_part_flat):
  nwp, m = den_part_flat.shape
  bm = 8192 if m > 8192 else m

  def body(dp_ref, o_ref):
    o_ref[...] = jnp.sum(dp_ref[...], axis=0, keepdims=True)

  return _tc_call(
      body, (pl.cdiv(m, bm),),
      [pl.BlockSpec((nwp, bm), lambda i: (0, i))],
      pl.BlockSpec((1, bm), lambda i: (0, i)),
      jax.ShapeDtypeStruct((1, m), jnp.float32),
      "tc_den_reduce")(den_part_flat)


def _node_update(num2, den, r, ts, qs, wt_l, wq_l, ones_exp):
  n = r.shape[0]
  hid = r.shape[1]
  bn = min(n, 2048)

  def body(num_ref, den_ref, r_ref, ts_ref, qs_ref, wt_ref, wq_ref,
           oe_ref, h_ref):
    num = jnp.concatenate([num_ref[0], num_ref[1]], axis=-1)
    den = den_ref[...] + jnp.float32(1e-9)
    den_e = jnp.dot(den, oe_ref[...], preferred_element_type=jnp.float32,
                    precision=lax.Precision.HIGHEST)
    v = num / den_e + r_ref[...] + ts_ref[...] * wt_ref[...] \
        + qs_ref[...] * wq_ref[...]
    h_ref[...] = jnp.where(v > 0, v, jnp.exp(jnp.minimum(v, 0.0)) - 1.0)

  return _tc_call(
      body, (pl.cdiv(n, bn),),
      [
          pl.BlockSpec((2, bn, hid // 2), lambda i: (0, i, 0)),
          pl.BlockSpec((bn, 8), lambda i: (i, 0)),
          pl.BlockSpec((bn, hid), lambda i: (i, 0)),
          pl.BlockSpec((bn, 1), lambda i: (i, 0)),
          pl.BlockSpec((bn, 1), lambda i: (i, 0)),
          pl.BlockSpec((1, hid), lambda i: (0, 0)),
          pl.BlockSpec((1, hid), lambda i: (0, 0)),
          pl.BlockSpec((8, hid), lambda i: (0, 0)),
      ],
      pl.BlockSpec((bn, hid), lambda i: (i, 0)),
      jax.ShapeDtypeStruct((n, hid), jnp.float32),
      "tc_node_update")(num2, den, r, ts, qs, wt_l, wq_l, ones_exp)


def _split_rows(h):
  n, hid = h.shape
  bn = min(n, 2048)

  def body(h_ref, o_ref):
    o_ref[0] = h_ref[:, :hid // 2]
    o_ref[1] = h_ref[:, hid // 2:]

  return _tc_call(
      body, (pl.cdiv(n, bn),),
      [pl.BlockSpec((bn, hid), lambda i: (i, 0))],
      pl.BlockSpec((2, bn, hid // 2), lambda i: (0, i, 0)),
      jax.ShapeDtypeStruct((2, n, hid // 2), jnp.float32),
      "tc_split")(h)


def _sum_pool(h, batch_col, g_cnt):
  n, hid = h.shape
  bn = min(n, 2048)

  def body(h_ref, b_ref, o_ref):
    @pl.when(pl.program_id(0) == 0)
    def _():
      o_ref[...] = jnp.zeros_like(o_ref)
    valid = n - pl.program_id(0) * bn
    rows = lax.broadcasted_iota(jnp.int32, (bn, g_cnt), 0)
    onehot = (b_ref[...] == lax.broadcasted_iota(jnp.int32, (bn, g_cnt), 1)
              ).astype(jnp.float32)
    onehot = jnp.where(rows < valid, onehot, 0.0)
    o_ref[...] += lax.dot_general(
        onehot, h_ref[...], (((0,), (0,)), ((), ())),
        preferred_element_type=jnp.float32,
        precision=lax.Precision.HIGHEST)

  return _tc_call(
      body, (pl.cdiv(n, bn),),
      [
          pl.BlockSpec((bn, hid), lambda i: (i, 0)),
          pl.BlockSpec((bn, 1), lambda i: (i, 0)),
      ],
      pl.BlockSpec((g_cnt, hid), lambda i: (0, 0)),
      jax.ShapeDtypeStruct((g_cnt, hid), jnp.float32),
      "tc_sum_pool")(h, batch_col)


def _readout(sum_pool, maxpart, temps, m1a, m1b, m1c, b1, m2, b2, m3, b3):
  g_cnt = sum_pool.shape[0]
  nwp = maxpart.shape[0]

  def body(sp_ref, mp_ref, t_ref, m1a_ref, m1b_ref, m1c_ref, b1_ref,
           m2_ref, b2_ref, m3_ref, b3_ref, o_ref):
    mp0 = mp_ref[0]
    mp1 = mp_ref[1]
    for k in range(2, nwp, 2):
      mp0 = jnp.maximum(mp0, mp_ref[k])
      mp1 = jnp.maximum(mp1, mp_ref[k + 1])
    mp = jnp.concatenate([mp0, mp1], axis=-1)
    mp = jnp.where(mp <= -1e29, 0.0, mp)
    u = jnp.dot(sp_ref[...], m1a_ref[...], preferred_element_type=jnp.float32)
    u += jnp.dot(mp, m1b_ref[...], preferred_element_type=jnp.float32)
    u += jnp.dot(t_ref[...], m1c_ref[...], preferred_element_type=jnp.float32)
    u += b1_ref[...]
    u = jnp.where(u > 0, u, jnp.exp(jnp.minimum(u, 0.0)) - 1.0)
    u = jnp.dot(u, m2_ref[...], preferred_element_type=jnp.float32) + b2_ref[...]
    u = jnp.where(u > 0, u, jnp.exp(jnp.minimum(u, 0.0)) - 1.0)
    o_ref[...] = jnp.dot(u, m3_ref[...],
                         preferred_element_type=jnp.float32) + b3_ref[...]

  hid = sum_pool.shape[1]
  return _tc_call(
      body, (1,),
      [
          pl.BlockSpec((g_cnt, hid), lambda i: (0, 0)),
          pl.BlockSpec((nwp, g_cnt, hid // 2), lambda i: (0, 0, 0)),
          pl.BlockSpec((g_cnt, 1), lambda i: (0, 0)),
          pl.BlockSpec((hid, 256), lambda i: (0, 0)),
          pl.BlockSpec((hid, 256), lambda i: (0, 0)),
          pl.BlockSpec((1, 256), lambda i: (0, 0)),
          pl.BlockSpec((1, 256), lambda i: (0, 0)),
          pl.BlockSpec((256, 256), lambda i: (0, 0)),
          pl.BlockSpec((1, 256), lambda i: (0, 0)),
          pl.BlockSpec((256, 1), lambda i: (0, 0)),
          pl.BlockSpec((1, 1), lambda i: (0, 0)),
      ],
      pl.BlockSpec((g_cnt, 1), lambda i: (0, 0)),
      jax.ShapeDtypeStruct((g_cnt, 1), jnp.float32),
      "tc_readout")(sum_pool, maxpart, temps, m1a, m1b, m1c, b1, m2, b2,
                    m3, b3)


def _block_diag_heads(a):
  h, dh = a.shape
  eye = jnp.eye(h, dtype=a.dtype)
  return (eye[:, None, :] * a[:, :, None]).reshape(h * dh, h)


def kernel(x, pos, edge_attr, temps, edge_index, triple_index, quadra_index,
           batch, W0, W12, Wres0, Wres12, A_src, A_dst, We, wd, wt, wq,
           M1, b1, M2, b2, M3, b3):
  n = x.shape[0]
  g_cnt = temps.shape[0]
  hid = W0.shape[1]
  i32 = jnp.int32

  src = edge_index[0].astype(i32)
  dst = edge_index[1].astype(i32)
  tix = triple_index[0].astype(i32)
  tjx = triple_index[1].astype(i32)
  tkx = triple_index[2].astype(i32)
  qix = quadra_index[0].astype(i32)
  qjx = quadra_index[1].astype(i32)
  qkx = quadra_index[2].astype(i32)
  qlx = quadra_index[3].astype(i32)
  batch_i = batch.astype(i32)
  pos3 = jnp.concatenate([pos[:, 0], pos[:, 1], pos[:, 2]])

  dist, ts_part, qs_part = _geom_call(pos3, src, dst, tix, tjx, tkx,
                                      qix, qjx, qkx, qlx)

  we_all = jnp.transpose(We, (1, 0, 2)).reshape(16, -1)
  wd_all = wd.reshape(1, -1)
  base = _prep_base(edge_attr, dist.reshape(-1, 1), we_all, wd_all)

  ones_exp = jnp.repeat(jnp.eye(8, dtype=jnp.float32), hid // 8, axis=1)

  ws = [W0, W12[0], W12[1]]
  wrs = [Wres0, Wres12[0], Wres12[1]]
  sa_mats = [jnp.concatenate(
      [_block_diag_heads(A_src[l]), _block_diag_heads(A_dst[l])], axis=1)
      for l in range(3)]

  h = x
  ts = qs = None
  for l in range(3):
    if l == 0:
      z2, stab, r, ts, qs = _layer_head(h, ws[l], wrs[l], sa_mats[l],
                                        extras=(ts_part, qs_part))
    else:
      z2, stab, r = _layer_head(h, ws[l], wrs[l], sa_mats[l])
    ex, den_part = _score_call(stab, base[l], src, dst)
    num2 = _agg_call(z2.reshape(2 * n, hid // 2), ex, src, dst)
    den = _den_reduce(den_part).reshape(n, 8)
    h = _node_update(num2.reshape(2, n, hid // 2), den, r, ts, qs,
                     wt[l].reshape(1, hid), wq[l].reshape(1, hid), ones_exp)

  h2 = _split_rows(h)
  maxpart = _maxpool_call(h2.reshape(2 * n, hid // 2), batch_i, g_cnt)
  sp = _sum_pool(h, batch_i.reshape(n, 1), g_cnt)
  out = _readout(sp, maxpart, temps, M1[:hid], M1[hid:2 * hid],
                 M1[2 * hid:], b1.reshape(1, -1), M2, b2.reshape(1, -1),
                 M3, b3.reshape(1, 1))
  return out


# speedup vs baseline: 26.2006x; 1.0958x over previous
import functools
from typing import Any

import jax
import jax.numpy as jnp
from jax import lax
from jax.experimental import pallas as pl
from jax.experimental.pallas import tpu as pltpu
from jax.experimental.pallas import tpu_sc as plsc

NC = 2
NS = 16
LN = 16
NW = NC * NS

_INTERPRET = False


def _mesh():
  return plsc.VectorSubcoreMesh(
      core_axis_name="c", subcore_axis_name="s", num_cores=NC,
      num_subcores=NS)


def _sc_params():
  return pltpu.CompilerParams(
      use_tc_tiling_on_sc=False, needs_layout_passes=False)


def _wid():
  return lax.axis_index("s") * NC + lax.axis_index("c")


def _iota16():
  return lax.iota(jnp.int32, 16)


def _splat(x):
  return jnp.full((16,), x, jnp.int32)


def _rsqrt(x):
  i = plsc.bitcast(x, jnp.int32)
  i = jnp.int32(0x5F3759DF) - lax.shift_right_arithmetic(i, 1)
  y = plsc.bitcast(i, jnp.float32)
  xh = x * jnp.float32(0.5)
  for _ in range(3):
    y = y * (jnp.float32(1.5) - xh * y * y)
  return y


def _sqrt16(x):
  return x * _rsqrt(x)


def _geom_body(n, e, t, q, c_sz,
               px, py, pz, src, dst, ti, tj, tk, qi, qj, qk, ql,
               dist_o, ts_part, qs_part,
               pxv, pyv, pzv, ts_acc, qs_acc, i0, i1, i2, i3, fb):
  w = _wid()
  it = _iota16()
  lane_masks = [it == j for j in range(16)]
  per = e // NW
  nch = per // c_sz
  ng = c_sz // 16

  pltpu.sync_copy(px, pxv)
  pltpu.sync_copy(py, pyv)
  pltpu.sync_copy(pz, pzv)

  def zero(ref, cnt):
    def zb(i, _):
      ref[pl.ds(i * 16, 16)] = jnp.zeros((16,), jnp.float32)
      return 0
    lax.fori_loop(0, cnt, zb, 0)

  zero(ts_acc, n // 16)
  zero(qs_acc, n // 16)

  def echunk(ci, _):
    base = w * per + ci * c_sz
    pltpu.sync_copy(src.at[pl.ds(base, c_sz)], i0)
    pltpu.sync_copy(dst.at[pl.ds(base, c_sz)], i1)

    for g in range(ng):
      s = i0[pl.ds(g * 16, 16)]
      d = i1[pl.ds(g * 16, 16)]
      dx = plsc.load_gather(pxv, [s]) - plsc.load_gather(pxv, [d])
      dy = plsc.load_gather(pyv, [s]) - plsc.load_gather(pyv, [d])
      dz = plsc.load_gather(pzv, [s]) - plsc.load_gather(pzv, [d])
      ss = dx * dx + dy * dy + dz * dz + jnp.float32(1e-8)
      fb[pl.ds(g * 16, 16)] = _sqrt16(ss)
    pltpu.sync_copy(fb, dist_o.at[pl.ds(base, c_sz)])
    return 0
  lax.fori_loop(0, nch, echunk, 0)

  def tchunk(ci, _):
    base = w * per + ci * c_sz
    pltpu.sync_copy(ti.at[pl.ds(base, c_sz)], i0)
    pltpu.sync_copy(tj.at[pl.ds(base, c_sz)], i1)
    pltpu.sync_copy(tk.at[pl.ds(base, c_sz)], i2)

    for g in range(ng):
      a = i0[pl.ds(g * 16, 16)]
      b = i1[pl.ds(g * 16, 16)]
      c = i2[pl.ds(g * 16, 16)]
      bx = plsc.load_gather(pxv, [b])
      by = plsc.load_gather(pyv, [b])
      bz = plsc.load_gather(pzv, [b])
      v1x = plsc.load_gather(pxv, [a]) - bx
      v1y = plsc.load_gather(pyv, [a]) - by
      v1z = plsc.load_gather(pzv, [a]) - bz
      v2x = plsc.load_gather(pxv, [c]) - bx
      v2y = plsc.load_gather(pyv, [c]) - by
      v2z = plsc.load_gather(pzv, [c]) - bz
      dot = v1x * v2x + v1y * v2y + v1z * v2z
      s1 = v1x * v1x + v1y * v1y + v1z * v1z
      s2 = v2x * v2x + v2y * v2y + v2z * v2z
      cos = dot / (_sqrt16(s1 * s2) + jnp.float32(1e-8))
      for lm in lane_masks:
        plsc.addupdate_scatter(ts_acc, [b], cos, mask=lm)
    return 0
  lax.fori_loop(0, nch, tchunk, 0)

  def qchunk(ci, _):
    base = w * per + ci * c_sz
    pltpu.sync_copy(qi.at[pl.ds(base, c_sz)], i0)
    pltpu.sync_copy(qj.at[pl.ds(base, c_sz)], i1)
    pltpu.sync_copy(qk.at[pl.ds(base, c_sz)], i2)
    pltpu.sync_copy(ql.at[pl.ds(base, c_sz)], i3)

    for g in range(ng):
      a = i0[pl.ds(g * 16, 16)]
      b = i1[pl.ds(g * 16, 16)]
      c = i2[pl.ds(g * 16, 16)]
      d = i3[pl.ds(g * 16, 16)]
      ax = plsc.load_gather(pxv, [a]); ay = plsc.load_gather(pyv, [a]); az = plsc.load_gather(pzv, [a])
      bx = plsc.load_gather(pxv, [b]); by = plsc.load_gather(pyv, [b]); bz = plsc.load_gather(pzv, [b])
      cx = plsc.load_gather(pxv, [c]); cy = plsc.load_gather(pyv, [c]); cz = plsc.load_gather(pzv, [c])
      ex = plsc.load_gather(pxv, [d]); ey = plsc.load_gather(pyv, [d]); ez = plsc.load_gather(pzv, [d])
      b1x = bx - ax; b1y = by - ay; b1z = bz - az
      b2x = cx - bx; b2y = cy - by; b2z = cz - bz
      b3x = ex - cx; b3y = ey - cy; b3z = ez - cz
      n1x = b1y * b2z - b1z * b2y
      n1y = b1z * b2x - b1x * b2z
      n1z = b1x * b2y - b1y * b2x
      n2x = b2y * b3z - b2z * b3y
      n2y = b2z * b3x - b2x * b3z
      n2z = b2x * b3y - b2y * b3x
      dot = n1x * n2x + n1y * n2y + n1z * n2z
      s1 = n1x * n1x + n1y * n1y + n1z * n1z
      s2 = n2x * n2x + n2y * n2y + n2z * n2z
      cos = dot / (_sqrt16(s1 * s2) + jnp.float32(1e-8))
      for lm in lane_masks:
        plsc.addupdate_scatter(qs_acc, [b], cos, mask=lm)
    return 0
  lax.fori_loop(0, nch, qchunk, 0)

  pltpu.sync_copy(ts_acc, ts_part.at[w])
  pltpu.sync_copy(qs_acc, qs_part.at[w])


def _geom_call(pos3, src, dst, tix, tjx, tkx, qix, qjx, qkx, qlx):
  n = pos3.shape[0] // 3
  e = src.shape[0]
  c_sz = 80 if e % (NW * 80) == 0 else 16
  px = pos3[:n]
  py = pos3[n:2 * n]
  pz = pos3[2 * n:]
  body = functools.partial(_geom_body, n, e, tix.shape[0], qix.shape[0],
                           c_sz)
  f = pl.kernel(
      body,
      out_type=(
          jax.ShapeDtypeStruct((e,), jnp.float32),
          jax.ShapeDtypeStruct((NW, n), jnp.float32),
          jax.ShapeDtypeStruct((NW, n), jnp.float32),
      ),
      mesh=_mesh(),
      compiler_params=_sc_params(),
      scratch_types=(
          pltpu.VMEM((n,), jnp.float32),
          pltpu.VMEM((n,), jnp.float32),
          pltpu.VMEM((n,), jnp.float32),
          pltpu.VMEM((n,), jnp.float32),
          pltpu.VMEM((n,), jnp.float32),
          pltpu.VMEM((c_sz,), jnp.int32),
          pltpu.VMEM((c_sz,), jnp.int32),
          pltpu.VMEM((c_sz,), jnp.int32),
          pltpu.VMEM((c_sz,), jnp.int32),
          pltpu.VMEM((c_sz,), jnp.float32),
      ),
      interpret=_INTERPRET,
      name="sc_geom",
  )
  return f(px, py, pz, src, dst, tix, tjx, tkx, qix, qjx, qkx, qlx)


def _score_body(n, e, c_sz,
                stab, base_l, src, dst,
                ex_o, den_part,
                srows, drows, bbuf, exb, sidx, didx, den_acc, sem):
  w = _wid()
  it = _iota16()
  lo8 = it < 8
  hi8 = it >= 8
  pair = lax.shift_right_logical(it, 3)
  h8 = jnp.bitwise_and(it, 7)
  per = e // NW
  nch = per // c_sz

  def zero(i, _):
    den_acc[pl.ds(i * 16, 16)] = jnp.zeros((16,), jnp.float32)
    return 0
  lax.fori_loop(0, n * 8 // 16, zero, 0)

  h8p8 = h8 + 8

  def chunk(ci, _):
    base = w * per + ci * c_sz
    d0 = pltpu.async_copy(src.at[pl.ds(base, c_sz)], sidx, sem)
    d1 = pltpu.async_copy(dst.at[pl.ds(base, c_sz)], didx, sem)
    d2 = pltpu.async_copy(base_l.at[pl.ds(base, c_sz)], bbuf, sem)
    d0.wait(); d1.wait(); d2.wait()
    g0 = pltpu.async_copy(stab.at[sidx], srows, sem)
    g1 = pltpu.async_copy(stab.at[didx], drows, sem)
    g0.wait(); g1.wait()

    for g in range(c_sz // 2):
      rp = pair + 2 * g
      sv = plsc.load_gather(srows, [rp, h8])
      dv = plsc.load_gather(drows, [rp, h8p8])
      bv = plsc.load_gather(bbuf, [rp, h8])
      sc = sv + dv + bv
      sc = jnp.maximum(sc, sc * jnp.float32(0.2))
      ev = jnp.exp(sc)
      plsc.store_scatter(exb, [rp, h8], ev)
      dg = plsc.load_gather(didx, [rp])
      tgt = dg * 8 + h8
      plsc.addupdate_scatter(den_acc, [tgt], ev, mask=lo8)
      plsc.addupdate_scatter(den_acc, [tgt], ev, mask=hi8)
    pltpu.sync_copy(exb, ex_o.at[pl.ds(base, c_sz)])
    return 0
  lax.fori_loop(0, nch, chunk, 0)
  pltpu.sync_copy(den_acc, den_part.at[w])


def _score_call(stab, base_l, src, dst):
  n = stab.shape[0]
  e = src.shape[0]
  c_sz = 80 if e % (NW * 80) == 0 else 16
  body = functools.partial(_score_body, n, e, c_sz)
  f = pl.kernel(
      body,
      out_type=(
          jax.ShapeDtypeStruct((e, 8), jnp.float32),
          jax.ShapeDtypeStruct((NW, n * 8), jnp.float32),
      ),
      mesh=_mesh(),
      compiler_params=_sc_params(),
      scratch_types=(
          pltpu.VMEM((c_sz, 16), jnp.float32),
          pltpu.VMEM((c_sz, 16), jnp.float32),
          pltpu.VMEM((c_sz, 8), jnp.float32),
          pltpu.VMEM((c_sz, 8), jnp.float32),
          pltpu.VMEM((c_sz,), jnp.int32),
          pltpu.VMEM((c_sz,), jnp.int32),
          pltpu.VMEM((n * 8,), jnp.float32),
          pltpu.SemaphoreType.DMA,
      ),
      interpret=_INTERPRET,
      name="sc_score",
  )
  return f(stab, base_l, src, dst)


def _agg_body(n, e, c_sz, sub,
              z2, ex, src2, dst2,
              num_o,
              zrows, exb, didx2, zidx2, zzero, acc, gsem, ssem):
  c = lax.axis_index("c")
  s = lax.axis_index("s")
  per = e // NS
  nsub = per // sub
  spc = c_sz // sub
  nch = per // c_sz
  rows_t = n // NS
  zch = zzero.shape[0]
  nzch = rows_t // zch
  it = _iota16()

  zv = jnp.zeros((16,), jnp.float32)
  def zb(i, _):
    for j in range(8):
      plsc.store_scatter(zzero, [_splat(i), it + j * 16], zv)
    return 0
  lax.fori_loop(0, zch, zb, 0)
  def zdma(m, _):
    pltpu.sync_copy(zzero, acc.at[pl.ds(s * rows_t + m * zch, zch)])
    return 0
  lax.fori_loop(0, nzch, zdma, 0)
  plsc.subcore_barrier()

  col_splats = [_splat(0) + c * 4 + jh for jh in range(4)]

  def chunk(ci, _):
    base = s * per + ci * c_sz
    r0 = s * nsub + ci * spc
    pltpu.sync_copy(src2.at[pl.ds(r0, spc)], zidx2)
    pltpu.sync_copy(dst2.at[pl.ds(r0, spc)], didx2)
    for j in range(spc):
      for k in range(sub // 16):
        zidx2[j, pl.ds(k * 16, 16)] = zidx2[j, pl.ds(k * 16, 16)] + c * n
    descs = [
        pltpu.async_copy(z2.at[zidx2.at[j]],
                         zrows.at[pl.ds(j * sub, sub)], gsem)
        for j in range(spc)
    ]
    pltpu.sync_copy(ex.at[pl.ds(base * 8, c_sz * 8)], exb)
    for d in descs:
      d.wait()

    def edge(k, _):
      ks = _splat(k * 8)
      for jh in range(4):
        hv = plsc.load_gather(exb, [ks + col_splats[jh]])
        for jj in range(2):
          j = jh * 2 + jj
          zrows[k, pl.ds(j * 16, 16)] = zrows[k, pl.ds(j * 16, 16)] * hv
      return 0
    lax.fori_loop(0, c_sz, edge, 0)

    sdescs = [
        pltpu.async_copy(zrows.at[pl.ds(j * sub, sub)],
                         acc.at[didx2.at[j]], ssem, add=True)
        for j in range(spc)
    ]
    for d in sdescs:
      d.wait()
    return 0
  lax.fori_loop(0, nch, chunk, 0)
  plsc.subcore_barrier()

  def dump(m, _):
    off = s * rows_t + m * zch
    pltpu.sync_copy(acc.at[pl.ds(off, zch)],
                    num_o.at[pl.ds(c * n + off, zch)])
    return 0
  lax.fori_loop(0, nzch, dump, 0)


def _agg_call(z2, ex, src, dst):
  n = z2.shape[0] // 2
  e = src.shape[0]
  sub = 80 if e % (NS * 80) == 0 else 16
  c_sz = 2 * sub if e % (NS * 2 * sub) == 0 else sub
  rows_t = n // NS
  zch = 25 if rows_t % 25 == 0 else rows_t
  nsub = e // NS // sub
  src2 = src.reshape(NS * nsub, sub)
  dst2 = dst.reshape(NS * nsub, sub)
  body = functools.partial(_agg_body, n, e, c_sz, sub)
  f = pl.kernel(
      body,
      out_type=jax.ShapeDtypeStruct((2 * n, 128), jnp.float32),
      mesh=_mesh(),
      compiler_params=_sc_params(),
      scratch_types=(
          pltpu.VMEM((c_sz, 128), jnp.float32),
          pltpu.VMEM((c_sz * 8,), jnp.float32),
          pltpu.VMEM((c_sz // sub, sub), jnp.int32),
          pltpu.VMEM((c_sz // sub, sub), jnp.int32),
          pltpu.VMEM((zch, 128), jnp.float32),
          pltpu.VMEM_SHARED((n, 128), jnp.float32),
          pltpu.SemaphoreType.DMA,
          pltpu.SemaphoreType.DMA,
      ),
      interpret=_INTERPRET,
      name="sc_agg",
  )
  return f(z2, ex.reshape(-1), src2, dst2)


def _maxpool_body(n, g_cnt,
                  h2, batch,
                  maxpart,
                  acc, rows, bb, sem):
  c = lax.axis_index("c")
  s = lax.axis_index("s")
  w = _wid()
  it = _iota16()
  lane_masks = [it == j for j in range(16)]
  groups = n // 16
  gper = groups // NS
  rem = groups - gper * NS
  lo = s * gper + jnp.minimum(s, rem)
  cnt = gper + jnp.where(s < rem, 1, 0)

  ninf = jnp.full((16,), -1e30, jnp.float32)
  def ib(i, _):
    for j in range(8):
      plsc.store_scatter(acc, [_splat(i), it + j * 16], ninf)
    return 0
  lax.fori_loop(0, g_cnt, ib, 0)

  def grp(gg, _):
    pltpu.sync_copy(h2.at[pl.ds(c * n + gg * 16, 16)], rows)
    pltpu.sync_copy(batch.at[pl.ds(gg * 16, 16)], bb)
    bv = bb[:]
    for lane in range(16):
      gid = jnp.max(jnp.where(lane_masks[lane], bv, 0))
      for j in range(8):
        cp = it + j * 16
        cur = plsc.load_gather(acc, [_splat(gid), cp])
        row = plsc.load_gather(rows, [_splat(lane), cp])
        plsc.store_scatter(acc, [_splat(gid), cp], jnp.maximum(cur, row))
    return 0
  lax.fori_loop(lo, lo + cnt, grp, 0)
  pltpu.sync_copy(acc, maxpart.at[w])


def _maxpool_call(h2, batch, g_cnt):
  n = h2.shape[0] // 2
  body = functools.partial(_maxpool_body, n, g_cnt)
  f = pl.kernel(
      body,
      out_type=jax.ShapeDtypeStruct((NW, g_cnt, 128), jnp.float32),
      mesh=_mesh(),
      compiler_params=_sc_params(),
      scratch_types=(
          pltpu.VMEM((g_cnt, 128), jnp.float32),
          pltpu.VMEM((16, 128), jnp.float32),
          pltpu.VMEM((16,), jnp.int32),
          pltpu.SemaphoreType.DMA,
      ),
      interpret=_INTERPRET,
      name="sc_maxpool",
  )
  return f(h2, batch)


def _tc_call(body, grid, in_specs, out_specs, out_shape, name):
  return pl.pallas_call(
      body,
      grid=grid,
      in_specs=in_specs,
      out_specs=out_specs,
      out_shape=out_shape,
      interpret=_INTERPRET,
      name=name,
  )


def _prep_base(edge_attr, dist, we_all, wd_all):
  e = edge_attr.shape[0]
  be = min(e, 2048)
  nl = we_all.shape[1] // 8

  def body(ea_ref, d_ref, we_ref, wd_ref, o_ref):
    v = jnp.dot(ea_ref[...], we_ref[...],
                preferred_element_type=jnp.float32)
    v = v + d_ref[...] * wd_ref[...]
    for l in range(nl):
      o_ref[l] = v[:, 8 * l:8 * l + 8]

  return _tc_call(
      body, (pl.cdiv(e, be),),
      [
          pl.BlockSpec((be, 16), lambda i: (i, 0)),
          pl.BlockSpec((be, 1), lambda i: (i, 0)),
          pl.BlockSpec((16, 8 * nl), lambda i: (0, 0)),
          pl.BlockSpec((1, 8 * nl), lambda i: (0, 0)),
      ],
      pl.BlockSpec((nl, be, 8), lambda i: (0, i, 0)),
      jax.ShapeDtypeStruct((nl, e, 8), jnp.float32),
      "tc_prep_base")(edge_attr, dist, we_all, wd_all)


def _layer_head(h, w_mat, wr_mat, sa_mat, extras=None):
  n, din = h.shape
  bn = min(n, 2048)
  hid = w_mat.shape[1]
  with_parts = extras is not None

  def body(*refs):
    if with_parts:
      h_ref, w_ref, wr_ref, sa_ref, tp_ref, qp_ref, ones_ref, \
          z2_ref, s_ref, r_ref, ts_ref, qs_ref = refs
    else:
      h_ref, w_ref, wr_ref, sa_ref, z2_ref, s_ref, r_ref = refs
    z = jnp.dot(h_ref[...], w_ref[...], preferred_element_type=jnp.float32)
    z2_ref[0] = z[:, :hid // 2]
    z2_ref[1] = z[:, hid // 2:]
    s_ref[...] = jnp.dot(z, sa_ref[...], preferred_element_type=jnp.float32,
                         precision=lax.Precision.HIGHEST)
    r_ref[...] = jnp.dot(h_ref[...], wr_ref[...],
                         preferred_element_type=jnp.float32)
    if with_parts:
      ts_ref[...] = lax.dot_general(
          tp_ref[...], ones_ref[...], (((0,), (0,)), ((), ())),
          preferred_element_type=jnp.float32,
          precision=lax.Precision.HIGHEST)
      qs_ref[...] = lax.dot_general(
          qp_ref[...], ones_ref[...], (((0,), (0,)), ((), ())),
          preferred_element_type=jnp.float32,
          precision=lax.Precision.HIGHEST)

  in_specs = [
      pl.BlockSpec((bn, din), lambda i: (i, 0)),
      pl.BlockSpec((din, hid), lambda i: (0, 0)),
      pl.BlockSpec((din, hid), lambda i: (0, 0)),
      pl.BlockSpec((hid, 16), lambda i: (0, 0)),
  ]
  args = [h, w_mat, wr_mat, sa_mat]
  out_specs = [
      pl.BlockSpec((2, bn, hid // 2), lambda i: (0, i, 0)),
      pl.BlockSpec((bn, 16), lambda i: (i, 0)),
      pl.BlockSpec((bn, hid), lambda i: (i, 0)),
  ]
  out_shape = [
      jax.ShapeDtypeStruct((2, n, hid // 2), jnp.float32),
      jax.ShapeDtypeStruct((n, 16), jnp.float32),
      jax.ShapeDtypeStruct((n, hid), jnp.float32),
  ]
  if with_parts:
    ts_part, qs_part = extras
    nwp = ts_part.shape[0]
    in_specs += [
        pl.BlockSpec((nwp, bn), lambda i: (0, i)),
        pl.BlockSpec((nwp, bn), lambda i: (0, i)),
        pl.BlockSpec((nwp, 1), lambda i: (0, 0)),
    ]
    args += [ts_part, qs_part, jnp.ones((nwp, 1), jnp.float32)]
    out_specs += [
        pl.BlockSpec((bn, 1), lambda i: (i, 0)),
        pl.BlockSpec((bn, 1), lambda i: (i, 0)),
    ]
    out_shape += [
        jax.ShapeDtypeStruct((n, 1), jnp.float32),
        jax.ShapeDtypeStruct((n, 1), jnp.float32),
    ]
  return _tc_call(body, (pl.cdiv(n, bn),), in_specs, out_specs, out_shape,
                  "tc_layer_head")(*args)


def _den_reduce(den_part_flat):
  nwp, m = den_part_flat.shape
  bm = 8192 if m > 8192 else m

  def body(dp_ref, o_ref):
    o_ref[...] = jnp.sum(dp_ref[...], axis=0, keepdims=True)

  return _tc_call(
      body, (pl.cdiv(m, bm),),
      [pl.BlockSpec((nwp, bm), lambda i: (0, i))],
      pl.BlockSpec((1, bm), lambda i: (0, i)),
      jax.ShapeDtypeStruct((1, m), jnp.float32),
      "tc_den_reduce")(den_part_flat)


def _node_update(num2, den, r, ts, qs, wt_l, wq_l, ones_exp):
  n = r.shape[0]
  hid = r.shape[1]
  bn = min(n, 2048)

  def body(num_ref, den_ref, r_ref, ts_ref, qs_ref, wt_ref, wq_ref,
           oe_ref, h_ref):
    num = jnp.concatenate([num_ref[0], num_ref[1]], axis=-1)
    den = den_ref[...] + jnp.float32(1e-9)
    den_e = jnp.dot(den, oe_ref[...], preferred_element_type=jnp.float32,
                    precision=lax.Precision.HIGHEST)
    v = num / den_e + r_ref[...] + ts_ref[...] * wt_ref[...] \
        + qs_ref[...] * wq_ref[...]
    h_ref[...] = jnp.where(v > 0, v, jnp.exp(jnp.minimum(v, 0.0)) - 1.0)

  return _tc_call(
      body, (pl.cdiv(n, bn),),
      [
          pl.BlockSpec((2, bn, hid // 2), lambda i: (0, i, 0)),
          pl.BlockSpec((bn, 8), lambda i: (i, 0)),
          pl.BlockSpec((bn, hid), lambda i: (i, 0)),
          pl.BlockSpec((bn, 1), lambda i: (i, 0)),
          pl.BlockSpec((bn, 1), lambda i: (i, 0)),
          pl.BlockSpec((1, hid), lambda i: (0, 0)),
          pl.BlockSpec((1, hid), lambda i: (0, 0)),
          pl.BlockSpec((8, hid), lambda i: (0, 0)),
      ],
      pl.BlockSpec((bn, hid), lambda i: (i, 0)),
      jax.ShapeDtypeStruct((n, hid), jnp.float32),
      "tc_node_update")(num2, den, r, ts, qs, wt_l, wq_l, ones_exp)


def _split_rows(h):
  n, hid = h.shape
  bn = min(n, 2048)

  def body(h_ref, o_ref):
    o_ref[0] = h_ref[:, :hid // 2]
    o_ref[1] = h_ref[:, hid // 2:]

  return _tc_call(
      body, (pl.cdiv(n, bn),),
      [pl.BlockSpec((bn, hid), lambda i: (i, 0))],
      pl.BlockSpec((2, bn, hid // 2), lambda i: (0, i, 0)),
      jax.ShapeDtypeStruct((2, n, hid // 2), jnp.float32),
      "tc_split")(h)


def _sum_pool(h, batch_col, g_cnt):
  n, hid = h.shape
  bn = min(n, 2048)

  def body(h_ref, b_ref, o_ref):
    @pl.when(pl.program_id(0) == 0)
    def _():
      o_ref[...] = jnp.zeros_like(o_ref)
    valid = n - pl.program_id(0) * bn
    rows = lax.broadcasted_iota(jnp.int32, (bn, g_cnt), 0)
    onehot = (b_ref[...] == lax.broadcasted_iota(jnp.int32, (bn, g_cnt), 1)
              ).astype(jnp.float32)
    onehot = jnp.where(rows < valid, onehot, 0.0)
    o_ref[...] += lax.dot_general(
        onehot, h_ref[...], (((0,), (0,)), ((), ())),
        preferred_element_type=jnp.float32,
        precision=lax.Precision.HIGHEST)

  return _tc_call(
      body, (pl.cdiv(n, bn),),
      [
          pl.BlockSpec((bn, hid), lambda i: (i, 0)),
          pl.BlockSpec((bn, 1), lambda i: (i, 0)),
      ],
      pl.BlockSpec((g_cnt, hid), lambda i: (0, 0)),
      jax.ShapeDtypeStruct((g_cnt, hid), jnp.float32),
      "tc_sum_pool")(h, batch_col)


def _readout(sum_pool, maxpart, temps, m1a, m1b, m1c, b1, m2, b2, m3, b3):
  g_cnt = sum_pool.shape[0]
  nwp = maxpart.shape[0]

  def body(sp_ref, mp_ref, t_ref, m1a_ref, m1b_ref, m1c_ref, b1_ref,
           m2_ref, b2_ref, m3_ref, b3_ref, o_ref):
    mp0 = mp_ref[0]
    mp1 = mp_ref[1]
    for k in range(2, nwp, 2):
      mp0 = jnp.maximum(mp0, mp_ref[k])
      mp1 = jnp.maximum(mp1, mp_ref[k + 1])
    mp = jnp.concatenate([mp0, mp1], axis=-1)
    mp = jnp.where(mp <= -1e29, 0.0, mp)
    u = jnp.dot(sp_ref[...], m1a_ref[...], preferred_element_type=jnp.float32)
    u += jnp.dot(mp, m1b_ref[...], preferred_element_type=jnp.float32)
    u += jnp.dot(t_ref[...], m1c_ref[...], preferred_element_type=jnp.float32)
    u += b1_ref[...]
    u = jnp.where(u > 0, u, jnp.exp(jnp.minimum(u, 0.0)) - 1.0)
    u = jnp.dot(u, m2_ref[...], preferred_element_type=jnp.float32) + b2_ref[...]
    u = jnp.where(u > 0, u, jnp.exp(jnp.minimum(u, 0.0)) - 1.0)
    o_ref[...] = jnp.dot(u, m3_ref[...],
                         preferred_element_type=jnp.float32) + b3_ref[...]

  hid = sum_pool.shape[1]
  return _tc_call(
      body, (1,),
      [
          pl.BlockSpec((g_cnt, hid), lambda i: (0, 0)),
          pl.BlockSpec((nwp, g_cnt, hid // 2), lambda i: (0, 0, 0)),
          pl.BlockSpec((g_cnt, 1), lambda i: (0, 0)),
          pl.BlockSpec((hid, 256), lambda i: (0, 0)),
          pl.BlockSpec((hid, 256), lambda i: (0, 0)),
          pl.BlockSpec((1, 256), lambda i: (0, 0)),
          pl.BlockSpec((1, 256), lambda i: (0, 0)),
          pl.BlockSpec((256, 256), lambda i: (0, 0)),
          pl.BlockSpec((1, 256), lambda i: (0, 0)),
          pl.BlockSpec((256, 1), lambda i: (0, 0)),
          pl.BlockSpec((1, 1), lambda i: (0, 0)),
      ],
      pl.BlockSpec((g_cnt, 1), lambda i: (0, 0)),
      jax.ShapeDtypeStruct((g_cnt, 1), jnp.float32),
      "tc_readout")(sum_pool, maxpart, temps, m1a, m1b, m1c, b1, m2, b2,
                    m3, b3)


def _block_diag_heads(a):
  h, dh = a.shape
  eye = jnp.eye(h, dtype=a.dtype)
  return (eye[:, None, :] * a[:, :, None]).reshape(h * dh, h)


def kernel(x, pos, edge_attr, temps, edge_index, triple_index, quadra_index,
           batch, W0, W12, Wres0, Wres12, A_src, A_dst, We, wd, wt, wq,
           M1, b1, M2, b2, M3, b3):
  n = x.shape[0]
  g_cnt = temps.shape[0]
  hid = W0.shape[1]
  i32 = jnp.int32

  src = edge_index[0].astype(i32)
  dst = edge_index[1].astype(i32)
  tix = triple_index[0].astype(i32)
  tjx = triple_index[1].astype(i32)
  tkx = triple_index[2].astype(i32)
  qix = quadra_index[0].astype(i32)
  qjx = quadra_index[1].astype(i32)
  qkx = quadra_index[2].astype(i32)
  qlx = quadra_index[3].astype(i32)
  batch_i = batch.astype(i32)
  pos3 = jnp.concatenate([pos[:, 0], pos[:, 1], pos[:, 2]])

  dist, ts_part, qs_part = _geom_call(pos3, src, dst, tix, tjx, tkx,
                                      qix, qjx, qkx, qlx)

  we_all = jnp.transpose(We, (1, 0, 2)).reshape(16, -1)
  wd_all = wd.reshape(1, -1)
  base = _prep_base(edge_attr, dist.reshape(-1, 1), we_all, wd_all)

  ones_exp = jnp.repeat(jnp.eye(8, dtype=jnp.float32), hid // 8, axis=1)

  ws = [W0, W12[0], W12[1]]
  wrs = [Wres0, Wres12[0], Wres12[1]]
  sa_mats = [jnp.concatenate(
      [_block_diag_heads(A_src[l]), _block_diag_heads(A_dst[l])], axis=1)
      for l in range(3)]

  h = x
  ts = qs = None
  for l in range(3):
    if l == 0:
      z2, stab, r, ts, qs = _layer_head(h, ws[l], wrs[l], sa_mats[l],
                                        extras=(ts_part, qs_part))
    else:
      z2, stab, r = _layer_head(h, ws[l], wrs[l], sa_mats[l])
    ex, den_part = _score_call(stab, base[l], src, dst)
    num2 = _agg_call(z2.reshape(2 * n, hid // 2), ex, src, dst)
    den = _den_reduce(den_part).reshape(n, 8)
    h = _node_update(num2.reshape(2, n, hid // 2), den, r, ts, qs,
                     wt[l].reshape(1, hid), wq[l].reshape(1, hid), ones_exp)

  h2 = _split_rows(h)
  maxpart = _maxpool_call(h2.reshape(2 * n, hid // 2), batch_i, g_cnt)
  sp = _sum_pool(h, batch_i.reshape(n, 1), g_cnt)
  out = _readout(sp, maxpart, temps, M1[:hid], M1[hid:2 * hid],
                 M1[2 * hid:], b1.reshape(1, -1), M2, b2.reshape(1, -1),
                 M3, b3.reshape(1, 1))
  return out


# speedup vs baseline: 27.7511x; 1.0592x over previous
import functools
from typing import Any

import jax
import jax.numpy as jnp
from jax import lax
from jax.experimental import pallas as pl
from jax.experimental.pallas import tpu as pltpu
from jax.experimental.pallas import tpu_sc as plsc

NC = 2
NS = 16
LN = 16
NW = NC * NS

_INTERPRET = False


def _mesh():
  return plsc.VectorSubcoreMesh(
      core_axis_name="c", subcore_axis_name="s", num_cores=NC,
      num_subcores=NS)


def _sc_params():
  return pltpu.CompilerParams(
      use_tc_tiling_on_sc=False, needs_layout_passes=False)


def _wid():
  return lax.axis_index("s") * NC + lax.axis_index("c")


def _iota16():
  return lax.iota(jnp.int32, 16)


def _splat(x):
  return jnp.full((16,), x, jnp.int32)


def _rsqrt(x):
  i = plsc.bitcast(x, jnp.int32)
  i = jnp.int32(0x5F3759DF) - lax.shift_right_arithmetic(i, 1)
  y = plsc.bitcast(i, jnp.float32)
  xh = x * jnp.float32(0.5)
  for _ in range(3):
    y = y * (jnp.float32(1.5) - xh * y * y)
  return y


def _sqrt16(x):
  return x * _rsqrt(x)


def _geom_body(n, e, t, q, c_sz,
               px, py, pz, src, dst, ti, tj, tk, qi, qj, qk, ql,
               dist_o, ts_part, qs_part,
               pxv, pyv, pzv, ts_acc, qs_acc, i0, i1, i2, i3, fb, gsem):
  w = _wid()
  it = _iota16()
  lane_masks = [it == j for j in range(16)]
  per = e // NW
  nch = per // c_sz
  ng = c_sz // 16

  pltpu.sync_copy(px, pxv)
  pltpu.sync_copy(py, pyv)
  pltpu.sync_copy(pz, pzv)

  def zero(ref, cnt):
    def zb(i, _):
      ref[pl.ds(i * 16, 16)] = jnp.zeros((16,), jnp.float32)
      return 0
    lax.fori_loop(0, cnt, zb, 0)

  zero(ts_acc, n // 16)
  zero(qs_acc, n // 16)

  def echunk(ci, _):
    base = w * per + ci * c_sz
    d0 = pltpu.async_copy(src.at[pl.ds(base, c_sz)], i0, gsem)
    d1 = pltpu.async_copy(dst.at[pl.ds(base, c_sz)], i1, gsem)
    d0.wait(); d1.wait()

    for g in range(ng):
      s = i0[pl.ds(g * 16, 16)]
      d = i1[pl.ds(g * 16, 16)]
      dx = plsc.load_gather(pxv, [s]) - plsc.load_gather(pxv, [d])
      dy = plsc.load_gather(pyv, [s]) - plsc.load_gather(pyv, [d])
      dz = plsc.load_gather(pzv, [s]) - plsc.load_gather(pzv, [d])
      ss = dx * dx + dy * dy + dz * dz + jnp.float32(1e-8)
      fb[pl.ds(g * 16, 16)] = _sqrt16(ss)
    pltpu.sync_copy(fb, dist_o.at[pl.ds(base, c_sz)])
    return 0
  lax.fori_loop(0, nch, echunk, 0)

  def tchunk(ci, _):
    base = w * per + ci * c_sz
    d0 = pltpu.async_copy(ti.at[pl.ds(base, c_sz)], i0, gsem)
    d1 = pltpu.async_copy(tj.at[pl.ds(base, c_sz)], i1, gsem)
    d2 = pltpu.async_copy(tk.at[pl.ds(base, c_sz)], i2, gsem)
    d0.wait(); d1.wait(); d2.wait()

    for g in range(ng):
      a = i0[pl.ds(g * 16, 16)]
      b = i1[pl.ds(g * 16, 16)]
      c = i2[pl.ds(g * 16, 16)]
      bx = plsc.load_gather(pxv, [b])
      by = plsc.load_gather(pyv, [b])
      bz = plsc.load_gather(pzv, [b])
      v1x = plsc.load_gather(pxv, [a]) - bx
      v1y = plsc.load_gather(pyv, [a]) - by
      v1z = plsc.load_gather(pzv, [a]) - bz
      v2x = plsc.load_gather(pxv, [c]) - bx
      v2y = plsc.load_gather(pyv, [c]) - by
      v2z = plsc.load_gather(pzv, [c]) - bz
      dot = v1x * v2x + v1y * v2y + v1z * v2z
      s1 = v1x * v1x + v1y * v1y + v1z * v1z
      s2 = v2x * v2x + v2y * v2y + v2z * v2z
      cos = dot / (_sqrt16(s1 * s2) + jnp.float32(1e-8))
      for lm in lane_masks:
        plsc.addupdate_scatter(ts_acc, [b], cos, mask=lm)
    return 0
  lax.fori_loop(0, nch, tchunk, 0)

  def qchunk(ci, _):
    base = w * per + ci * c_sz
    d0 = pltpu.async_copy(qi.at[pl.ds(base, c_sz)], i0, gsem)
    d1 = pltpu.async_copy(qj.at[pl.ds(base, c_sz)], i1, gsem)
    d2 = pltpu.async_copy(qk.at[pl.ds(base, c_sz)], i2, gsem)
    d3 = pltpu.async_copy(ql.at[pl.ds(base, c_sz)], i3, gsem)
    d0.wait(); d1.wait(); d2.wait(); d3.wait()

    for g in range(ng):
      a = i0[pl.ds(g * 16, 16)]
      b = i1[pl.ds(g * 16, 16)]
      c = i2[pl.ds(g * 16, 16)]
      d = i3[pl.ds(g * 16, 16)]
      ax = plsc.load_gather(pxv, [a]); ay = plsc.load_gather(pyv, [a]); az = plsc.load_gather(pzv, [a])
      bx = plsc.load_gather(pxv, [b]); by = plsc.load_gather(pyv, [b]); bz = plsc.load_gather(pzv, [b])
      cx = plsc.load_gather(pxv, [c]); cy = plsc.load_gather(pyv, [c]); cz = plsc.load_gather(pzv, [c])
      ex = plsc.load_gather(pxv, [d]); ey = plsc.load_gather(pyv, [d]); ez = plsc.load_gather(pzv, [d])
      b1x = bx - ax; b1y = by - ay; b1z = bz - az
      b2x = cx - bx; b2y = cy - by; b2z = cz - bz
      b3x = ex - cx; b3y = ey - cy; b3z = ez - cz
      n1x = b1y * b2z - b1z * b2y
      n1y = b1z * b2x - b1x * b2z
      n1z = b1x * b2y - b1y * b2x
      n2x = b2y * b3z - b2z * b3y
      n2y = b2z * b3x - b2x * b3z
      n2z = b2x * b3y - b2y * b3x
      dot = n1x * n2x + n1y * n2y + n1z * n2z
      s1 = n1x * n1x + n1y * n1y + n1z * n1z
      s2 = n2x * n2x + n2y * n2y + n2z * n2z
      cos = dot / (_sqrt16(s1 * s2) + jnp.float32(1e-8))
      for lm in lane_masks:
        plsc.addupdate_scatter(qs_acc, [b], cos, mask=lm)
    return 0
  lax.fori_loop(0, nch, qchunk, 0)

  pltpu.sync_copy(ts_acc, ts_part.at[w])
  pltpu.sync_copy(qs_acc, qs_part.at[w])


def _geom_call(pos3, src, dst, tix, tjx, tkx, qix, qjx, qkx, qlx):
  n = pos3.shape[0] // 3
  e = src.shape[0]
  c_sz = 80 if e % (NW * 80) == 0 else 16
  px = pos3[:n]
  py = pos3[n:2 * n]
  pz = pos3[2 * n:]
  body = functools.partial(_geom_body, n, e, tix.shape[0], qix.shape[0],
                           c_sz)
  f = pl.kernel(
      body,
      out_type=(
          jax.ShapeDtypeStruct((e,), jnp.float32),
          jax.ShapeDtypeStruct((NW, n), jnp.float32),
          jax.ShapeDtypeStruct((NW, n), jnp.float32),
      ),
      mesh=_mesh(),
      compiler_params=_sc_params(),
      scratch_types=(
          pltpu.VMEM((n,), jnp.float32),
          pltpu.VMEM((n,), jnp.float32),
          pltpu.VMEM((n,), jnp.float32),
          pltpu.VMEM((n,), jnp.float32),
          pltpu.VMEM((n,), jnp.float32),
          pltpu.VMEM((c_sz,), jnp.int32),
          pltpu.VMEM((c_sz,), jnp.int32),
          pltpu.VMEM((c_sz,), jnp.int32),
          pltpu.VMEM((c_sz,), jnp.int32),
          pltpu.VMEM((c_sz,), jnp.float32),
          pltpu.SemaphoreType.DMA,
      ),
      interpret=_INTERPRET,
      name="sc_geom",
  )
  return f(px, py, pz, src, dst, tix, tjx, tkx, qix, qjx, qkx, qlx)


def _score_body(n, e, c_sz,
                stab, base_l, src, dst,
                ex_o, den_part,
                srows, drows, bbuf, exb, sidx, didx, den_acc, sem):
  w = _wid()
  it = _iota16()
  lo8 = it < 8
  hi8 = it >= 8
  pair = lax.shift_right_logical(it, 3)
  h8 = jnp.bitwise_and(it, 7)
  per = e // NW
  nch = per // c_sz

  def zero(i, _):
    den_acc[pl.ds(i * 16, 16)] = jnp.zeros((16,), jnp.float32)
    return 0
  lax.fori_loop(0, n * 8 // 16, zero, 0)

  h8p8 = h8 + 8

  def chunk(ci, _):
    base = w * per + ci * c_sz
    d0 = pltpu.async_copy(src.at[pl.ds(base, c_sz)], sidx, sem)
    d1 = pltpu.async_copy(dst.at[pl.ds(base, c_sz)], didx, sem)
    d2 = pltpu.async_copy(base_l.at[pl.ds(base, c_sz)], bbuf, sem)
    d0.wait(); d1.wait(); d2.wait()
    g0 = pltpu.async_copy(stab.at[sidx], srows, sem)
    g1 = pltpu.async_copy(stab.at[didx], drows, sem)
    g0.wait(); g1.wait()

    for g in range(c_sz // 2):
      rp = pair + 2 * g
      sv = plsc.load_gather(srows, [rp, h8])
      dv = plsc.load_gather(drows, [rp, h8p8])
      bv = plsc.load_gather(bbuf, [rp, h8])
      sc = sv + dv + bv
      sc = jnp.maximum(sc, sc * jnp.float32(0.2))
      ev = jnp.exp(sc)
      plsc.store_scatter(exb, [rp, h8], ev)
      dg = plsc.load_gather(didx, [rp])
      tgt = dg * 8 + h8
      plsc.addupdate_scatter(den_acc, [tgt], ev, mask=lo8)
      plsc.addupdate_scatter(den_acc, [tgt], ev, mask=hi8)
    pltpu.sync_copy(exb, ex_o.at[pl.ds(base, c_sz)])
    return 0
  lax.fori_loop(0, nch, chunk, 0)
  pltpu.sync_copy(den_acc, den_part.at[w])


def _score_call(stab, base_l, src, dst):
  n = stab.shape[0]
  e = src.shape[0]
  c_sz = 80 if e % (NW * 80) == 0 else 16
  body = functools.partial(_score_body, n, e, c_sz)
  f = pl.kernel(
      body,
      out_type=(
          jax.ShapeDtypeStruct((e, 8), jnp.float32),
          jax.ShapeDtypeStruct((NW, n * 8), jnp.float32),
      ),
      mesh=_mesh(),
      compiler_params=_sc_params(),
      scratch_types=(
          pltpu.VMEM((c_sz, 16), jnp.float32),
          pltpu.VMEM((c_sz, 16), jnp.float32),
          pltpu.VMEM((c_sz, 8), jnp.float32),
          pltpu.VMEM((c_sz, 8), jnp.float32),
          pltpu.VMEM((c_sz,), jnp.int32),
          pltpu.VMEM((c_sz,), jnp.int32),
          pltpu.VMEM((n * 8,), jnp.float32),
          pltpu.SemaphoreType.DMA,
      ),
      interpret=_INTERPRET,
      name="sc_score",
  )
  return f(stab, base_l, src, dst)


def _agg_body(n, e, c_sz, sub,
              z2, ex, src2, dst2,
              num_o,
              zrows, exb, didx2, zidx2, zzero, acc, gsem, ssem):
  c = lax.axis_index("c")
  s = lax.axis_index("s")
  per = e // NS
  nsub = per // sub
  spc = c_sz // sub
  nch = per // c_sz
  rows_t = n // NS
  zch = zzero.shape[0]
  nzch = rows_t // zch
  it = _iota16()

  zv = jnp.zeros((16,), jnp.float32)
  def zb(i, _):
    for j in range(8):
      plsc.store_scatter(zzero, [_splat(i), it + j * 16], zv)
    return 0
  lax.fori_loop(0, zch, zb, 0)
  def zdma(m, _):
    pltpu.sync_copy(zzero, acc.at[pl.ds(s * rows_t + m * zch, zch)])
    return 0
  lax.fori_loop(0, nzch, zdma, 0)
  plsc.subcore_barrier()

  col_splats = [_splat(0) + c * 4 + jh for jh in range(4)]

  def chunk(ci, _):
    base = s * per + ci * c_sz
    r0 = s * nsub + ci * spc
    pltpu.sync_copy(src2.at[pl.ds(r0, spc)], zidx2)
    pltpu.sync_copy(dst2.at[pl.ds(r0, spc)], didx2)
    for j in range(spc):
      for k in range(sub // 16):
        zidx2[j, pl.ds(k * 16, 16)] = zidx2[j, pl.ds(k * 16, 16)] + c * n
    descs = [
        pltpu.async_copy(z2.at[zidx2.at[j]],
                         zrows.at[pl.ds(j * sub, sub)], gsem)
        for j in range(spc)
    ]
    pltpu.sync_copy(ex.at[pl.ds(base * 8, c_sz * 8)], exb)
    for d in descs:
      d.wait()

    def edge(k, _):
      ks = _splat(k * 8)
      for jh in range(4):
        hv = plsc.load_gather(exb, [ks + col_splats[jh]])
        for jj in range(2):
          j = jh * 2 + jj
          zrows[k, pl.ds(j * 16, 16)] = zrows[k, pl.ds(j * 16, 16)] * hv
      return 0
    lax.fori_loop(0, c_sz, edge, 0)

    sdescs = [
        pltpu.async_copy(zrows.at[pl.ds(j * sub, sub)],
                         acc.at[didx2.at[j]], ssem, add=True)
        for j in range(spc)
    ]
    for d in sdescs:
      d.wait()
    return 0
  lax.fori_loop(0, nch, chunk, 0)
  plsc.subcore_barrier()

  def dump(m, _):
    off = s * rows_t + m * zch
    pltpu.sync_copy(acc.at[pl.ds(off, zch)],
                    num_o.at[pl.ds(c * n + off, zch)])
    return 0
  lax.fori_loop(0, nzch, dump, 0)


def _agg_call(z2, ex, src, dst):
  n = z2.shape[0] // 2
  e = src.shape[0]
  sub = 80 if e % (NS * 80) == 0 else 16
  c_sz = 2 * sub if e % (NS * 2 * sub) == 0 else sub
  rows_t = n // NS
  zch = 25 if rows_t % 25 == 0 else rows_t
  nsub = e // NS // sub
  src2 = src.reshape(NS * nsub, sub)
  dst2 = dst.reshape(NS * nsub, sub)
  body = functools.partial(_agg_body, n, e, c_sz, sub)
  f = pl.kernel(
      body,
      out_type=jax.ShapeDtypeStruct((2 * n, 128), jnp.float32),
      mesh=_mesh(),
      compiler_params=_sc_params(),
      scratch_types=(
          pltpu.VMEM((c_sz, 128), jnp.float32),
          pltpu.VMEM((c_sz * 8,), jnp.float32),
          pltpu.VMEM((c_sz // sub, sub), jnp.int32),
          pltpu.VMEM((c_sz // sub, sub), jnp.int32),
          pltpu.VMEM((zch, 128), jnp.float32),
          pltpu.VMEM_SHARED((n, 128), jnp.float32),
          pltpu.SemaphoreType.DMA,
          pltpu.SemaphoreType.DMA,
      ),
      interpret=_INTERPRET,
      name="sc_agg",
  )
  return f(z2, ex.reshape(-1), src2, dst2)


def _maxpool_body(n, g_cnt,
                  h2, batch,
                  maxpart,
                  acc, rows, bb, sem):
  c = lax.axis_index("c")
  s = lax.axis_index("s")
  w = _wid()
  it = _iota16()
  lane_masks = [it == j for j in range(16)]
  groups = n // 16
  gper = groups // NS
  rem = groups - gper * NS
  lo = s * gper + jnp.minimum(s, rem)
  cnt = gper + jnp.where(s < rem, 1, 0)

  ninf = jnp.full((16,), -1e30, jnp.float32)
  def ib(i, _):
    for j in range(8):
      plsc.store_scatter(acc, [_splat(i), it + j * 16], ninf)
    return 0
  lax.fori_loop(0, g_cnt, ib, 0)

  def grp(gg, _):
    pltpu.sync_copy(h2.at[pl.ds(c * n + gg * 16, 16)], rows)
    pltpu.sync_copy(batch.at[pl.ds(gg * 16, 16)], bb)
    bv = bb[:]
    for lane in range(16):
      gid = jnp.max(jnp.where(lane_masks[lane], bv, 0))
      for j in range(8):
        cp = it + j * 16
        cur = plsc.load_gather(acc, [_splat(gid), cp])
        row = plsc.load_gather(rows, [_splat(lane), cp])
        plsc.store_scatter(acc, [_splat(gid), cp], jnp.maximum(cur, row))
    return 0
  lax.fori_loop(lo, lo + cnt, grp, 0)
  pltpu.sync_copy(acc, maxpart.at[w])


def _maxpool_call(h2, batch, g_cnt):
  n = h2.shape[0] // 2
  body = functools.partial(_maxpool_body, n, g_cnt)
  f = pl.kernel(
      body,
      out_type=jax.ShapeDtypeStruct((NW, g_cnt, 128), jnp.float32),
      mesh=_mesh(),
      compiler_params=_sc_params(),
      scratch_types=(
          pltpu.VMEM((g_cnt, 128), jnp.float32),
          pltpu.VMEM((16, 128), jnp.float32),
          pltpu.VMEM((16,), jnp.int32),
          pltpu.SemaphoreType.DMA,
      ),
      interpret=_INTERPRET,
      name="sc_maxpool",
  )
  return f(h2, batch)


def _tc_call(body, grid, in_specs, out_specs, out_shape, name):
  return pl.pallas_call(
      body,
      grid=grid,
      in_specs=in_specs,
      out_specs=out_specs,
      out_shape=out_shape,
      interpret=_INTERPRET,
      name=name,
  )


def _prep_base(edge_attr, dist, we_all, wd_all):
  e = edge_attr.shape[0]
  be = min(e, 2048)
  nl = we_all.shape[1] // 8

  def body(ea_ref, d_ref, we_ref, wd_ref, o_ref):
    v = jnp.dot(ea_ref[...], we_ref[...],
                preferred_element_type=jnp.float32)
    v = v + d_ref[...] * wd_ref[...]
    for l in range(nl):
      o_ref[l] = v[:, 8 * l:8 * l + 8]

  return _tc_call(
      body, (pl.cdiv(e, be),),
      [
          pl.BlockSpec((be, 16), lambda i: (i, 0)),
          pl.BlockSpec((be, 1), lambda i: (i, 0)),
          pl.BlockSpec((16, 8 * nl), lambda i: (0, 0)),
          pl.BlockSpec((1, 8 * nl), lambda i: (0, 0)),
      ],
      pl.BlockSpec((nl, be, 8), lambda i: (0, i, 0)),
      jax.ShapeDtypeStruct((nl, e, 8), jnp.float32),
      "tc_prep_base")(edge_attr, dist, we_all, wd_all)


def _layer_head(h, w_mat, wr_mat, sa_mat, extras=None):
  n, din = h.shape
  bn = min(n, 2048)
  hid = w_mat.shape[1]
  with_parts = extras is not None

  def body(*refs):
    if with_parts:
      h_ref, w_ref, wr_ref, sa_ref, tp_ref, qp_ref, ones_ref, \
          z2_ref, s_ref, r_ref, ts_ref, qs_ref = refs
    else:
      h_ref, w_ref, wr_ref, sa_ref, z2_ref, s_ref, r_ref = refs
    z = jnp.dot(h_ref[...], w_ref[...], preferred_element_type=jnp.float32)
    z2_ref[0] = z[:, :hid // 2]
    z2_ref[1] = z[:, hid // 2:]
    s_ref[...] = jnp.dot(z, sa_ref[...], preferred_element_type=jnp.float32,
                         precision=lax.Precision.HIGHEST)
    r_ref[...] = jnp.dot(h_ref[...], wr_ref[...],
                         preferred_element_type=jnp.float32)
    if with_parts:
      ts_ref[...] = lax.dot_general(
          tp_ref[...], ones_ref[...], (((0,), (0,)), ((), ())),
          preferred_element_type=jnp.float32,
          precision=lax.Precision.HIGHEST)
      qs_ref[...] = lax.dot_general(
          qp_ref[...], ones_ref[...], (((0,), (0,)), ((), ())),
          preferred_element_type=jnp.float32,
          precision=lax.Precision.HIGHEST)

  in_specs = [
      pl.BlockSpec((bn, din), lambda i: (i, 0)),
      pl.BlockSpec((din, hid), lambda i: (0, 0)),
      pl.BlockSpec((din, hid), lambda i: (0, 0)),
      pl.BlockSpec((hid, 16), lambda i: (0, 0)),
  ]
  args = [h, w_mat, wr_mat, sa_mat]
  out_specs = [
      pl.BlockSpec((2, bn, hid // 2), lambda i: (0, i, 0)),
      pl.BlockSpec((bn, 16), lambda i: (i, 0)),
      pl.BlockSpec((bn, hid), lambda i: (i, 0)),
  ]
  out_shape = [
      jax.ShapeDtypeStruct((2, n, hid // 2), jnp.float32),
      jax.ShapeDtypeStruct((n, 16), jnp.float32),
      jax.ShapeDtypeStruct((n, hid), jnp.float32),
  ]
  if with_parts:
    ts_part, qs_part = extras
    nwp = ts_part.shape[0]
    in_specs += [
        pl.BlockSpec((nwp, bn), lambda i: (0, i)),
        pl.BlockSpec((nwp, bn), lambda i: (0, i)),
        pl.BlockSpec((nwp, 1), lambda i: (0, 0)),
    ]
    args += [ts_part, qs_part, jnp.ones((nwp, 1), jnp.float32)]
    out_specs += [
        pl.BlockSpec((bn, 1), lambda i: (i, 0)),
        pl.BlockSpec((bn, 1), lambda i: (i, 0)),
    ]
    out_shape += [
        jax.ShapeDtypeStruct((n, 1), jnp.float32),
        jax.ShapeDtypeStruct((n, 1), jnp.float32),
    ]
  return _tc_call(body, (pl.cdiv(n, bn),), in_specs, out_specs, out_shape,
                  "tc_layer_head")(*args)


def _den_reduce(den_part_flat):
  nwp, m = den_part_flat.shape
  bm = 8192 if m > 8192 else m

  def body(dp_ref, o_ref):
    o_ref[...] = jnp.sum(dp_ref[...], axis=0, keepdims=True)

  return _tc_call(
      body, (pl.cdiv(m, bm),),
      [pl.BlockSpec((nwp, bm), lambda i: (0, i))],
      pl.BlockSpec((1, bm), lambda i: (0, i)),
      jax.ShapeDtypeStruct((1, m), jnp.float32),
      "tc_den_reduce")(den_part_flat)


def _node_update(num2, den, r, ts, qs, wt_l, wq_l, ones_exp):
  n = r.shape[0]
  hid = r.shape[1]
  bn = min(n, 2048)

  def body(num_ref, den_ref, r_ref, ts_ref, qs_ref, wt_ref, wq_ref,
           oe_ref, h_ref):
    num = jnp.concatenate([num_ref[0], num_ref[1]], axis=-1)
    den = den_ref[...] + jnp.float32(1e-9)
    den_e = jnp.dot(den, oe_ref[...], preferred_element_type=jnp.float32,
                    precision=lax.Precision.HIGHEST)
    v = num / den_e + r_ref[...] + ts_ref[...] * wt_ref[...] \
        + qs_ref[...] * wq_ref[...]
    h_ref[...] = jnp.where(v > 0, v, jnp.exp(jnp.minimum(v, 0.0)) - 1.0)

  return _tc_call(
      body, (pl.cdiv(n, bn),),
      [
          pl.BlockSpec((2, bn, hid // 2), lambda i: (0, i, 0)),
          pl.BlockSpec((bn, 8), lambda i: (i, 0)),
          pl.BlockSpec((bn, hid), lambda i: (i, 0)),
          pl.BlockSpec((bn, 1), lambda i: (i, 0)),
          pl.BlockSpec((bn, 1), lambda i: (i, 0)),
          pl.BlockSpec((1, hid), lambda i: (0, 0)),
          pl.BlockSpec((1, hid), lambda i: (0, 0)),
          pl.BlockSpec((8, hid), lambda i: (0, 0)),
      ],
      pl.BlockSpec((bn, hid), lambda i: (i, 0)),
      jax.ShapeDtypeStruct((n, hid), jnp.float32),
      "tc_node_update")(num2, den, r, ts, qs, wt_l, wq_l, ones_exp)


def _split_rows(h):
  n, hid = h.shape
  bn = min(n, 2048)

  def body(h_ref, o_ref):
    o_ref[0] = h_ref[:, :hid // 2]
    o_ref[1] = h_ref[:, hid // 2:]

  return _tc_call(
      body, (pl.cdiv(n, bn),),
      [pl.BlockSpec((bn, hid), lambda i: (i, 0))],
      pl.BlockSpec((2, bn, hid // 2), lambda i: (0, i, 0)),
      jax.ShapeDtypeStruct((2, n, hid // 2), jnp.float32),
      "tc_split")(h)


def _sum_pool(h, batch_col, g_cnt):
  n, hid = h.shape
  bn = min(n, 2048)

  def body(h_ref, b_ref, o_ref):
    @pl.when(pl.program_id(0) == 0)
    def _():
      o_ref[...] = jnp.zeros_like(o_ref)
    valid = n - pl.program_id(0) * bn
    rows = lax.broadcasted_iota(jnp.int32, (bn, g_cnt), 0)
    onehot = (b_ref[...] == lax.broadcasted_iota(jnp.int32, (bn, g_cnt), 1)
              ).astype(jnp.float32)
    onehot = jnp.where(rows < valid, onehot, 0.0)
    o_ref[...] += lax.dot_general(
        onehot, h_ref[...], (((0,), (0,)), ((), ())),
        preferred_element_type=jnp.float32,
        precision=lax.Precision.HIGHEST)

  return _tc_call(
      body, (pl.cdiv(n, bn),),
      [
          pl.BlockSpec((bn, hid), lambda i: (i, 0)),
          pl.BlockSpec((bn, 1), lambda i: (i, 0)),
      ],
      pl.BlockSpec((g_cnt, hid), lambda i: (0, 0)),
      jax.ShapeDtypeStruct((g_cnt, hid), jnp.float32),
      "tc_sum_pool")(h, batch_col)


def _readout(sum_pool, maxpart, temps, m1a, m1b, m1c, b1, m2, b2, m3, b3):
  g_cnt = sum_pool.shape[0]
  nwp = maxpart.shape[0]

  def body(sp_ref, mp_ref, t_ref, m1a_ref, m1b_ref, m1c_ref, b1_ref,
           m2_ref, b2_ref, m3_ref, b3_ref, o_ref):
    mp0 = mp_ref[0]
    mp1 = mp_ref[1]
    for k in range(2, nwp, 2):
      mp0 = jnp.maximum(mp0, mp_ref[k])
      mp1 = jnp.maximum(mp1, mp_ref[k + 1])
    mp = jnp.concatenate([mp0, mp1], axis=-1)
    mp = jnp.where(mp <= -1e29, 0.0, mp)
    u = jnp.dot(sp_ref[...], m1a_ref[...], preferred_element_type=jnp.float32)
    u += jnp.dot(mp, m1b_ref[...], preferred_element_type=jnp.float32)
    u += jnp.dot(t_ref[...], m1c_ref[...], preferred_element_type=jnp.float32)
    u += b1_ref[...]
    u = jnp.where(u > 0, u, jnp.exp(jnp.minimum(u, 0.0)) - 1.0)
    u = jnp.dot(u, m2_ref[...], preferred_element_type=jnp.float32) + b2_ref[...]
    u = jnp.where(u > 0, u, jnp.exp(jnp.minimum(u, 0.0)) - 1.0)
    o_ref[...] = jnp.dot(u, m3_ref[...],
                         preferred_element_type=jnp.float32) + b3_ref[...]

  hid = sum_pool.shape[1]
  return _tc_call(
      body, (1,),
      [
          pl.BlockSpec((g_cnt, hid), lambda i: (0, 0)),
          pl.BlockSpec((nwp, g_cnt, hid // 2), lambda i: (0, 0, 0)),
          pl.BlockSpec((g_cnt, 1), lambda i: (0, 0)),
          pl.BlockSpec((hid, 256), lambda i: (0, 0)),
          pl.BlockSpec((hid, 256), lambda i: (0, 0)),
          pl.BlockSpec((1, 256), lambda i: (0, 0)),
          pl.BlockSpec((1, 256), lambda i: (0, 0)),
          pl.BlockSpec((256, 256), lambda i: (0, 0)),
          pl.BlockSpec((1, 256), lambda i: (0, 0)),
          pl.BlockSpec((256, 1), lambda i: (0, 0)),
          pl.BlockSpec((1, 1), lambda i: (0, 0)),
      ],
      pl.BlockSpec((g_cnt, 1), lambda i: (0, 0)),
      jax.ShapeDtypeStruct((g_cnt, 1), jnp.float32),
      "tc_readout")(sum_pool, maxpart, temps, m1a, m1b, m1c, b1, m2, b2,
                    m3, b3)


def _block_diag_heads(a):
  h, dh = a.shape
  eye = jnp.eye(h, dtype=a.dtype)
  return (eye[:, None, :] * a[:, :, None]).reshape(h * dh, h)


def kernel(x, pos, edge_attr, temps, edge_index, triple_index, quadra_index,
           batch, W0, W12, Wres0, Wres12, A_src, A_dst, We, wd, wt, wq,
           M1, b1, M2, b2, M3, b3):
  n = x.shape[0]
  g_cnt = temps.shape[0]
  hid = W0.shape[1]
  i32 = jnp.int32

  src = edge_index[0].astype(i32)
  dst = edge_index[1].astype(i32)
  tix = triple_index[0].astype(i32)
  tjx = triple_index[1].astype(i32)
  tkx = triple_index[2].astype(i32)
  qix = quadra_index[0].astype(i32)
  qjx = quadra_index[1].astype(i32)
  qkx = quadra_index[2].astype(i32)
  qlx = quadra_index[3].astype(i32)
  batch_i = batch.astype(i32)
  pos3 = jnp.concatenate([pos[:, 0], pos[:, 1], pos[:, 2]])

  dist, ts_part, qs_part = _geom_call(pos3, src, dst, tix, tjx, tkx,
                                      qix, qjx, qkx, qlx)

  we_all = jnp.transpose(We, (1, 0, 2)).reshape(16, -1)
  wd_all = wd.reshape(1, -1)
  base = _prep_base(edge_attr, dist.reshape(-1, 1), we_all, wd_all)

  ones_exp = jnp.repeat(jnp.eye(8, dtype=jnp.float32), hid // 8, axis=1)

  ws = [W0, W12[0], W12[1]]
  wrs = [Wres0, Wres12[0], Wres12[1]]
  sa_mats = [jnp.concatenate(
      [_block_diag_heads(A_src[l]), _block_diag_heads(A_dst[l])], axis=1)
      for l in range(3)]

  h = x
  ts = qs = None
  for l in range(3):
    if l == 0:
      z2, stab, r, ts, qs = _layer_head(h, ws[l], wrs[l], sa_mats[l],
                                        extras=(ts_part, qs_part))
    else:
      z2, stab, r = _layer_head(h, ws[l], wrs[l], sa_mats[l])
    ex, den_part = _score_call(stab, base[l], src, dst)
    num2 = _agg_call(z2.reshape(2 * n, hid // 2), ex, src, dst)
    den = _den_reduce(den_part).reshape(n, 8)
    h = _node_update(num2.reshape(2, n, hid // 2), den, r, ts, qs,
                     wt[l].reshape(1, hid), wq[l].reshape(1, hid), ones_exp)

  h2 = _split_rows(h)
  maxpart = _maxpool_call(h2.reshape(2 * n, hid // 2), batch_i, g_cnt)
  sp = _sum_pool(h, batch_i.reshape(n, 1), g_cnt)
  out = _readout(sp, maxpart, temps, M1[:hid], M1[hid:2 * hid],
                 M1[2 * hid:], b1.reshape(1, -1), M2, b2.reshape(1, -1),
                 M3, b3.reshape(1, 1))
  return out
